# baseline (speedup 1.0000x reference)
import functools

import jax
import jax.numpy as jnp
from jax import lax
from jax.experimental import pallas as pl
from jax.experimental.pallas import tpu as pltpu
from jax.experimental.pallas import tpu_sc as plsc

B = 4
N = 8192
P = 1024
R1, R2 = 0.1, 0.2
S1, S2 = 16, 32
CF = 64
CIN = CF + 3
CMID = 32
COUT = 128
EPS = 1e-5
M1 = B * P * S1
M2 = B * P * S2
NSUB = 32
CPW = P // (NSUB // B)
NSTEP = N // 16


def _fps_body(x_ref, y_ref, z_ref, xr_ref, o_ref, dists_ref):
    gidx = (lax.broadcasted_iota(jnp.int32, (B, 8, N // 8), 1) * (N // 8)
            + lax.broadcasted_iota(jnp.int32, (B, 8, N // 8), 2))

    dists_ref[...] = jnp.full((B, 8, N // 8), 1e10, jnp.float32)

    def fetch(iv):
        rows = [xr_ref[b, pl.ds(iv[b, 0, 0], 1), :] for b in range(B)]
        packed = jnp.concatenate(rows, axis=1)
        lx = jnp.concatenate([r[0:1, 0:1] for r in rows], 0).reshape(B, 1, 1)
        ly = jnp.concatenate([r[0:1, 1:2] for r in rows], 0).reshape(B, 1, 1)
        lz = jnp.concatenate([r[0:1, 2:3] for r in rows], 0).reshape(B, 1, 1)
        return packed, lx, ly, lz

    packed, lx, ly, lz = fetch(jnp.zeros((B, 1, 1), jnp.int32))
    o_ref[0:1, :] = packed

    def body(i, carry):
        lx, ly, lz = carry
        dx = x_ref[...] - lx
        dy = y_ref[...] - ly
        dz = z_ref[...] - lz
        d2 = dx * dx + dy * dy + dz * dz
        dm = jnp.minimum(dists_ref[...], d2)
        dists_ref[...] = dm
        mx = jnp.max(jnp.max(dm, axis=2, keepdims=True), axis=1,
                     keepdims=True)
        cand = jnp.where(dm == mx, gidx, N)
        iv = jnp.min(jnp.min(cand, axis=2, keepdims=True), axis=1,
                     keepdims=True)
        packed, lx, ly, lz = fetch(iv)
        o_ref[pl.ds(i, 1), :] = packed
        return (lx, ly, lz)

    lax.fori_loop(1, P, body, (lx, ly, lz))


def _run_fps(xt, xyz):
    x = xt[:, 0, :].reshape(B, 8, N // 8)
    y = xt[:, 1, :].reshape(B, 8, N // 8)
    z = xt[:, 2, :].reshape(B, 8, N // 8)
    xr = jnp.concatenate([xyz, jnp.zeros((B, N, 1), jnp.float32)], axis=-1)
    out = pl.pallas_call(
        _fps_body,
        out_shape=jax.ShapeDtypeStruct((P, 16), jnp.float32),
        scratch_shapes=[pltpu.VMEM((B, 8, N // 8), jnp.float32)],
    )(x, y, z, xr)
    return out


def _bq_body(xp_hbm, yp_hbm, zp_hbm, cxp_hbm, cyp_hbm, czp_hbm,
             i1_hbm, g1_hbm, i2_hbm, g2_hbm,
             xv, yv, zv, cxv, cyv, czv, o1i, o2i, o1g, o2g,
             t1i, t1d, t2i, t2d, cntv, d2b):
    w = lax.axis_index("c") * 16 + lax.axis_index("s")
    b = w // (NSUB // B)
    p0 = (w % (NSUB // B)) * CPW
    base = b * N

    pltpu.sync_copy(xp_hbm.at[b], xv)
    pltpu.sync_copy(yp_hbm.at[b], yv)
    pltpu.sync_copy(zp_hbm.at[b], zv)
    pltpu.sync_copy(cxp_hbm.at[b, pl.ds(p0, CPW)], cxv.at[pl.ds(0, CPW)])
    pltpu.sync_copy(cyp_hbm.at[b, pl.ds(p0, CPW)], cyv.at[pl.ds(0, CPW)])
    pltpu.sync_copy(czp_hbm.at[b, pl.ds(p0, CPW)], czv.at[pl.ds(0, CPW)])

    lane = lax.broadcasted_iota(jnp.int32, (16,), 0)
    r1sq = jnp.float32(R1 * R1)
    r2sq = jnp.float32(R2 * R2)

    def centroid(c, _):
        cx = cxv[pl.ds(c, 16)][0]
        cy = cyv[pl.ds(c, 16)][0]
        cz = czv[pl.ds(c, 16)][0]
        x0 = xv[pl.ds(0, 16)][0]
        y0 = yv[pl.ds(0, 16)][0]
        z0 = zv[pl.ds(0, 16)][0]
        d20 = ((x0 - cx) * (x0 - cx) + (y0 - cy) * (y0 - cy)
               + (z0 - cz) * (z0 - cz))
        t1i[pl.ds(0, 16)] = jnp.full((16,), base, jnp.int32)
        t1d[pl.ds(0, 16)] = jnp.full((16,), d20, jnp.float32)
        t2i[pl.ds(0, 16)] = jnp.full((16,), base, jnp.int32)
        t2d[pl.ds(0, 16)] = jnp.full((16,), d20, jnp.float32)

        zero16 = jnp.zeros((16,), jnp.int32)
        cntv[pl.ds(0, 16)] = zero16
        cntv[pl.ds(16, 16)] = zero16
        cntv[pl.ds(32, 16)] = zero16
        cntv[pl.ds(48, 16)] = zero16

        def branch(off4, tbi, tbd, coff, doff, rsq, cap):
            cv = cntv[pl.ds(coff, 16)]
            for k in range(4):
                d2 = d2b[pl.ds(16 * k, 16)]
                gi = (base + off4 + 16 * k) + lane
                m = (d2 < rsq) & (cv < cap)
                v = m.astype(jnp.int32)
                inc = plsc.cumsum(v)
                plsc.store_scatter(tbi, [cv + (inc - v)], gi, mask=m)
                plsc.store_scatter(tbd, [cv + (inc - v)], d2, mask=m)
                cv = cv + plsc.all_reduce_population_count(m)
            cntv[pl.ds(coff, 16)] = cv
            cntv[pl.ds(doff, 16)] = (cv >= cap).astype(jnp.int32)

        def chunk(jj, _):
            off4 = jj * 64
            d1 = cntv[pl.ds(32, 16)][0]
            d2f = cntv[pl.ds(48, 16)][0]

            @pl.when(d1 + d2f < 2)
            def _():
                for k in range(4):
                    xs = xv[pl.ds(off4 + 16 * k, 16)]
                    ys = yv[pl.ds(off4 + 16 * k, 16)]
                    zs = zv[pl.ds(off4 + 16 * k, 16)]
                    dx = xs - cx
                    dy = ys - cy
                    dz = zs - cz
                    d2b[pl.ds(16 * k, 16)] = dx * dx + dy * dy + dz * dz

            @pl.when(d1 == 0)
            def _():
                branch(off4, t1i, t1d, 0, 32, r1sq, S1)

            @pl.when(d2f == 0)
            def _():
                branch(off4, t2i, t2d, 16, 48, r2sq, S2)

            return 0

        lax.fori_loop(0, NSTEP // 4, chunk, 0)
        c1 = jnp.minimum(cntv[pl.ds(0, 16)][0], S1)
        c2 = jnp.minimum(cntv[pl.ds(16, 16)][0], S2)

        cc = jnp.full((16,), c, jnp.int32)
        f_i = t1i[pl.ds(0, 16)][0]
        f_d = t1d[pl.ds(0, 16)][0]
        vi = jnp.where(lane >= c1, f_i, t1i[pl.ds(0, 16)])
        vd = jnp.where(lane >= c1, f_d, t1d[pl.ds(0, 16)])
        o1i[c, :] = vi
        loc = vi - base
        gx = plsc.load_gather(xv, [loc])
        gy = plsc.load_gather(yv, [loc])
        gz = plsc.load_gather(zv, [loc])
        plsc.store_scatter(o1g, [cc, lane, jnp.full((16,), 0, jnp.int32)], gx)
        plsc.store_scatter(o1g, [cc, lane, jnp.full((16,), 1, jnp.int32)], gy)
        plsc.store_scatter(o1g, [cc, lane, jnp.full((16,), 2, jnp.int32)], gz)
        plsc.store_scatter(o1g, [cc, lane, jnp.full((16,), 3, jnp.int32)], vd)
        f_i2 = t2i[pl.ds(0, 16)][0]
        f_d2 = t2d[pl.ds(0, 16)][0]
        for k in range(2):
            lk = lane + (16 * k)
            vi = jnp.where(lk >= c2, f_i2, t2i[pl.ds(16 * k, 16)])
            vd = jnp.where(lk >= c2, f_d2, t2d[pl.ds(16 * k, 16)])
            o2i[c, pl.ds(16 * k, 16)] = vi
            loc = vi - base
            gx = plsc.load_gather(xv, [loc])
            gy = plsc.load_gather(yv, [loc])
            gz = plsc.load_gather(zv, [loc])
            plsc.store_scatter(o2g, [cc, lk, jnp.full((16,), 0, jnp.int32)], gx)
            plsc.store_scatter(o2g, [cc, lk, jnp.full((16,), 1, jnp.int32)], gy)
            plsc.store_scatter(o2g, [cc, lk, jnp.full((16,), 2, jnp.int32)], gz)
            plsc.store_scatter(o2g, [cc, lk, jnp.full((16,), 3, jnp.int32)], vd)
        return 0

    lax.fori_loop(0, CPW, centroid, 0)

    pltpu.sync_copy(o1i, i1_hbm.at[b, pl.ds(p0, CPW)])
    pltpu.sync_copy(o2i, i2_hbm.at[b, pl.ds(p0, CPW)])
    pltpu.sync_copy(o1g, g1_hbm.at[b, pl.ds(p0, CPW)])
    pltpu.sync_copy(o2g, g2_hbm.at[b, pl.ds(p0, CPW)])


def _run_ball_query(planes, cplanes):
    mesh = plsc.VectorSubcoreMesh(core_axis_name="c", subcore_axis_name="s",
                                   num_cores=2, num_subcores=16)
    f = pl.kernel(
        _bq_body,
        out_type=[
            pltpu.HBM((B, P, S1), jnp.int32),
            pltpu.HBM((B, P, S1, 4), jnp.float32),
            pltpu.HBM((B, P, S2), jnp.int32),
            pltpu.HBM((B, P, S2, 4), jnp.float32),
        ],
        mesh=mesh,
        compiler_params=pltpu.CompilerParams(needs_layout_passes=False,
                                             use_tc_tiling_on_sc=False),
        scratch_types=[
            pltpu.VMEM((N,), jnp.float32),
            pltpu.VMEM((N,), jnp.float32),
            pltpu.VMEM((N,), jnp.float32),
            pltpu.VMEM((CPW + 16,), jnp.float32),
            pltpu.VMEM((CPW + 16,), jnp.float32),
            pltpu.VMEM((CPW + 16,), jnp.float32),
            pltpu.VMEM((CPW, S1), jnp.int32),
            pltpu.VMEM((CPW, S2), jnp.int32),
            pltpu.VMEM((CPW, S1, 4), jnp.float32),
            pltpu.VMEM((CPW, S2, 4), jnp.float32),
            pltpu.VMEM((48,), jnp.int32),
            pltpu.VMEM((48,), jnp.float32),
            pltpu.VMEM((48,), jnp.int32),
            pltpu.VMEM((48,), jnp.float32),
            pltpu.VMEM((64,), jnp.int32),
            pltpu.VMEM((64,), jnp.float32),
        ],
    )
    return f(*planes, *cplanes)


def _gather_body(table_hbm, i1_hbm, i2_hbm, g1_hbm, g2_hbm,
                 idxv, buf, sem):
    w = lax.axis_index("c") * 16 + lax.axis_index("s")

    def run(idx_hbm, out_hbm, rows_per_sub):
        nchunk = rows_per_sub // 128
        r0 = w * rows_per_sub

        def chunk(c, _):
            rb = r0 + c * 128
            pltpu.sync_copy(idx_hbm.at[pl.ds(rb, 128)], idxv)
            pltpu.async_copy(table_hbm.at[idxv], buf, sem).wait()
            pltpu.sync_copy(buf, out_hbm.at[pl.ds(rb, 128)])
            return 0

        lax.fori_loop(0, nchunk, chunk, 0)

    run(i1_hbm, g1_hbm, M1 // NSUB)
    run(i2_hbm, g2_hbm, M2 // NSUB)


def _run_gather(table, idx1f, idx2f):
    mesh = plsc.VectorSubcoreMesh(core_axis_name="c", subcore_axis_name="s",
                                   num_cores=2, num_subcores=16)
    f = pl.kernel(
        _gather_body,
        out_type=[
            pltpu.HBM((M1, CF), jnp.float32),
            pltpu.HBM((M2, CF), jnp.float32),
        ],
        mesh=mesh,
        compiler_params=pltpu.CompilerParams(needs_layout_passes=False,
                                             use_tc_tiling_on_sc=False),
        scratch_types=[
            pltpu.VMEM((128,), jnp.int32),
            pltpu.VMEM((128, CF), jnp.float32),
            pltpu.SemaphoreType.DMA,
        ],
    )
    return f(table, idx1f, idx2f)


def _geom_h1(geo, cent, w1t, S, bm):
    q = bm // S
    geo3 = geo.reshape(q, S, 4)
    raw = geo3[:, :, 0:3]
    d2 = geo3[:, :, 3:4]
    delta = raw - cent.reshape(q, 1, 3)
    xi = jnp.broadcast_to(geo3[:, 0:1, 0:3], (q, S, 3))
    dist = jnp.sqrt(d2 + 1e-12)
    h10 = jnp.concatenate([dist, xi, raw, delta], axis=-1).reshape(bm, 10)
    h1 = jnp.dot(h10, w1t, preferred_element_type=jnp.float32)
    return h1, delta


def _pass1_body(geo_ref, cent_ref, w1t_ref, acc_ref, *, S, bm):
    h1, _ = _geom_h1(geo_ref[...], cent_ref[...], w1t_ref[...], S, bm)

    @pl.when(pl.program_id(0) == 0)
    def _():
        acc_ref[...] = jnp.zeros((8, 128), jnp.float32)

    acc_ref[0:1, 0:CMID] += jnp.sum(h1, axis=0, keepdims=True)
    acc_ref[1:2, 0:CMID] += jnp.sum(h1 * h1, axis=0, keepdims=True)


def _pass2_body(geo_ref, cent_ref, feat_ref, w1t_ref, acc1_ref, w2pt_ref,
                b2p_ref, m_ref, accy_ref, *, S, bm, mbr):
    q = bm // S
    h1, delta = _geom_h1(geo_ref[...], cent_ref[...], w1t_ref[...], S, bm)
    s1 = acc1_ref[0:1, 0:CMID]
    q1 = acc1_ref[1:2, 0:CMID]
    mu1 = s1 * (1.0 / mbr)
    var1 = q1 * (1.0 / mbr) - mu1 * mu1
    rs1 = lax.rsqrt(var1 + EPS)
    h = jnp.maximum((h1 - mu1) * rs1, 0.0)
    h2 = jnp.dot(h, w2pt_ref[...], preferred_element_type=jnp.float32)
    h2 = h2 + b2p_ref[...]
    x = jnp.concatenate([feat_ref[...].reshape(q, S, CF), delta], axis=-1)
    y3 = h2.reshape(q, S, CIN) * x
    y2 = y3.reshape(bm, CIN)
    m = jnp.max(y3, axis=1)
    m_ref[...] = jnp.concatenate(
        [m, jnp.zeros((q, 128 - CIN), jnp.float32)], axis=-1)

    @pl.when(pl.program_id(0) == 0)
    def _():
        accy_ref[...] = jnp.zeros((8, 128), jnp.float32)

    accy_ref[0:1, 0:CIN] += jnp.sum(y2, axis=0, keepdims=True)
    accy_ref[1:2, 0:CIN] += jnp.sum(y2 * y2, axis=0, keepdims=True)


def _pass3_body(m1_ref, a1_ref, m2_ref, a2_ref, wcrt_ref, o1_ref, o2_ref):
    for m_ref, a_ref, o_ref, mbr in ((m1_ref, a1_ref, o1_ref, M1),
                                     (m2_ref, a2_ref, o2_ref, M2)):
        sy = a_ref[0:1, 0:CIN]
        qy = a_ref[1:2, 0:CIN]
        mu = sy * (1.0 / mbr)
        var = qy * (1.0 / mbr) - mu * mu
        rs = lax.rsqrt(var + EPS)
        x3 = jnp.maximum((m_ref[...][:, 0:CIN] - mu) * rs, 0.0)
        z = jnp.dot(x3, wcrt_ref[...], preferred_element_type=jnp.float32)
        muz = jnp.sum(z, axis=0, keepdims=True) * (1.0 / (B * P))
        varz = jnp.sum(z * z, axis=0, keepdims=True) * (1.0 / (B * P)) \
            - muz * muz
        o_ref[...] = jnp.maximum((z - muz) * lax.rsqrt(varz + EPS), 0.0)


def _run_branch(geo, cent, feat, w1t, w2pt, b2p, S, mbr):
    bm = 4096
    grid = (mbr // bm,)
    q = bm // S
    acc1 = pl.pallas_call(
        functools.partial(_pass1_body, S=S, bm=bm),
        grid=grid,
        in_specs=[
            pl.BlockSpec((bm, 4), lambda i: (i, 0)),
            pl.BlockSpec((q, 3), lambda i: (i, 0)),
            pl.BlockSpec((10, CMID), lambda i: (0, 0)),
        ],
        out_specs=pl.BlockSpec((8, 128), lambda i: (0, 0)),
        out_shape=jax.ShapeDtypeStruct((8, 128), jnp.float32),
    )(geo, cent, w1t)
    m, accy = pl.pallas_call(
        functools.partial(_pass2_body, S=S, bm=bm, mbr=mbr),
        grid=grid,
        in_specs=[
            pl.BlockSpec((bm, 4), lambda i: (i, 0)),
            pl.BlockSpec((q, 3), lambda i: (i, 0)),
            pl.BlockSpec((bm, CF), lambda i: (i, 0)),
            pl.BlockSpec((10, CMID), lambda i: (0, 0)),
            pl.BlockSpec((8, 128), lambda i: (0, 0)),
            pl.BlockSpec((CMID, CIN), lambda i: (0, 0)),
            pl.BlockSpec((1, CIN), lambda i: (0, 0)),
        ],
        out_specs=[
            pl.BlockSpec((q, 128), lambda i: (i, 0)),
            pl.BlockSpec((8, 128), lambda i: (0, 0)),
        ],
        out_shape=[
            jax.ShapeDtypeStruct((B * P, 128), jnp.float32),
            jax.ShapeDtypeStruct((8, 128), jnp.float32),
        ],
    )(geo, cent, feat, w1t, acc1, w2pt, b2p)
    return m, accy


def kernel(xyz, features, w1, b1, w2, b2, wcr, bcr):
    xt = jnp.transpose(xyz, (0, 2, 1))
    o = _run_fps(xt, xyz)
    op = o.reshape(P, B, 4)
    new_xyz = jnp.transpose(op[:, :, 0:3], (1, 0, 2))

    planes = (xt[:, 0], xt[:, 1], xt[:, 2])
    cplanes = (op[:, :, 0].T, op[:, :, 1].T, op[:, :, 2].T)
    idx1, geo1, idx2, geo2 = _run_ball_query(planes, cplanes)

    table = jnp.transpose(features, (0, 2, 1)).reshape(B * N, CF)
    feat1, feat2 = _run_gather(table, idx1.reshape(M1), idx2.reshape(M2))

    perm = jnp.concatenate(
        [jnp.arange(3, CIN, dtype=jnp.int32),
         jnp.arange(0, 3, dtype=jnp.int32)])
    w1t = jnp.transpose(w1)
    w2pt = jnp.transpose(w2[perm, :])
    b2p = b2[perm].reshape(1, CIN)
    wcrt = jnp.transpose(wcr[:, perm])

    cent = new_xyz.reshape(B * P, 3)
    m1, ay1 = _run_branch(geo1.reshape(M1, 4), cent, feat1,
                          w1t, w2pt, b2p, S1, M1)
    m2, ay2 = _run_branch(geo2.reshape(M2, 4), cent, feat2,
                          w1t, w2pt, b2p, S2, M2)

    o1, o2 = pl.pallas_call(
        _pass3_body,
        out_shape=[jax.ShapeDtypeStruct((B * P, COUT), jnp.float32)] * 2,
    )(m1, ay1, m2, ay2, wcrt)

    f1 = o1.reshape(B, P, COUT).transpose(0, 2, 1)
    f2 = o2.reshape(B, P, COUT).transpose(0, 2, 1)
    return (new_xyz, jnp.concatenate([f1, f2], axis=1))


# speedup vs baseline: 1.3914x; 1.3914x over previous
import functools

import jax
import jax.numpy as jnp
from jax import lax
from jax.experimental import pallas as pl
from jax.experimental.pallas import tpu as pltpu
from jax.experimental.pallas import tpu_sc as plsc

B = 4
N = 8192
P = 1024
R1, R2 = 0.1, 0.2
S1, S2 = 16, 32
CF = 64
CIN = CF + 3
CMID = 32
COUT = 128
EPS = 1e-5
M1 = B * P * S1
M2 = B * P * S2
NSUB = 32
CPW = P // (NSUB // B)
NSTEP = N // 16


def _fps_body(x_ref, y_ref, z_ref, xr_ref, o_ref, dists_ref):
    gidx = (lax.broadcasted_iota(jnp.int32, (B, 8, N // 8), 1) * (N // 8)
            + lax.broadcasted_iota(jnp.int32, (B, 8, N // 8), 2))

    dists_ref[...] = jnp.full((B, 8, N // 8), 1e10, jnp.float32)

    def fetch(iv):
        rows = [xr_ref[b, pl.ds(iv[b, 0, 0], 1), :] for b in range(B)]
        packed = jnp.concatenate(rows, axis=1)
        lx = jnp.concatenate([r[0:1, 0:1] for r in rows], 0).reshape(B, 1, 1)
        ly = jnp.concatenate([r[0:1, 1:2] for r in rows], 0).reshape(B, 1, 1)
        lz = jnp.concatenate([r[0:1, 2:3] for r in rows], 0).reshape(B, 1, 1)
        return packed, lx, ly, lz

    packed, lx, ly, lz = fetch(jnp.zeros((B, 1, 1), jnp.int32))
    o_ref[0:1, :] = packed

    def body(i, carry):
        lx, ly, lz = carry
        dx = x_ref[...] - lx
        dy = y_ref[...] - ly
        dz = z_ref[...] - lz
        d2 = dx * dx + dy * dy + dz * dz
        dm = jnp.minimum(dists_ref[...], d2)
        dists_ref[...] = dm
        mx = jnp.max(jnp.max(dm, axis=2, keepdims=True), axis=1,
                     keepdims=True)
        cand = jnp.where(dm == mx, gidx, N)
        iv = jnp.min(jnp.min(cand, axis=2, keepdims=True), axis=1,
                     keepdims=True)
        packed, lx, ly, lz = fetch(iv)
        o_ref[pl.ds(i, 1), :] = packed
        return (lx, ly, lz)

    lax.fori_loop(1, P, body, (lx, ly, lz))


def _run_fps(xt, xyz):
    x = xt[:, 0, :].reshape(B, 8, N // 8)
    y = xt[:, 1, :].reshape(B, 8, N // 8)
    z = xt[:, 2, :].reshape(B, 8, N // 8)
    xr = jnp.concatenate([xyz, jnp.zeros((B, N, 1), jnp.float32)], axis=-1)
    out = pl.pallas_call(
        _fps_body,
        out_shape=jax.ShapeDtypeStruct((P, 16), jnp.float32),
        scratch_shapes=[pltpu.VMEM((B, 8, N // 8), jnp.float32)],
    )(x, y, z, xr)
    return out


def _bq_body(xp_hbm, yp_hbm, zp_hbm, cxp_hbm, cyp_hbm, czp_hbm,
             i1_hbm, g1_hbm, i2_hbm, g2_hbm,
             xv, yv, zv, cxv, cyv, czv, o1i, o2i, o1g, o2g,
             t1i, t1d, t2i, t2d, cntv, d2b):
    w = lax.axis_index("c") * 16 + lax.axis_index("s")
    b = w // (NSUB // B)
    p0 = (w % (NSUB // B)) * CPW
    base = b * N

    pltpu.sync_copy(xp_hbm.at[b], xv)
    pltpu.sync_copy(yp_hbm.at[b], yv)
    pltpu.sync_copy(zp_hbm.at[b], zv)
    pltpu.sync_copy(cxp_hbm.at[b, pl.ds(p0, CPW)], cxv.at[pl.ds(0, CPW)])
    pltpu.sync_copy(cyp_hbm.at[b, pl.ds(p0, CPW)], cyv.at[pl.ds(0, CPW)])
    pltpu.sync_copy(czp_hbm.at[b, pl.ds(p0, CPW)], czv.at[pl.ds(0, CPW)])

    lane = lax.broadcasted_iota(jnp.int32, (16,), 0)
    r1sq = jnp.float32(R1 * R1)
    r2sq = jnp.float32(R2 * R2)

    def centroid(c, _):
        cx = cxv[pl.ds(c, 16)][0]
        cy = cyv[pl.ds(c, 16)][0]
        cz = czv[pl.ds(c, 16)][0]
        x0 = xv[pl.ds(0, 16)][0]
        y0 = yv[pl.ds(0, 16)][0]
        z0 = zv[pl.ds(0, 16)][0]
        d20 = ((x0 - cx) * (x0 - cx) + (y0 - cy) * (y0 - cy)
               + (z0 - cz) * (z0 - cz))
        t1i[pl.ds(0, 16)] = jnp.full((16,), base, jnp.int32)
        t1d[pl.ds(0, 16)] = jnp.full((16,), d20, jnp.float32)
        t2i[pl.ds(0, 16)] = jnp.full((16,), base, jnp.int32)
        t2d[pl.ds(0, 16)] = jnp.full((16,), d20, jnp.float32)

        zero16 = jnp.zeros((16,), jnp.int32)
        cntv[pl.ds(0, 16)] = zero16
        cntv[pl.ds(16, 16)] = zero16
        cntv[pl.ds(32, 16)] = zero16
        cntv[pl.ds(48, 16)] = zero16

        def branch(d2s, off4, tbi, tbd, coff, doff, rsq, cap):
            cv = cntv[pl.ds(coff, 16)]
            for k in range(4):
                d2 = d2s[k]
                gi = (base + off4 + 16 * k) + lane
                m = (d2 < rsq) & (cv < cap)
                v = m.astype(jnp.int32)
                inc = plsc.cumsum(v)
                plsc.store_scatter(tbi, [cv + (inc - v)], gi, mask=m)
                plsc.store_scatter(tbd, [cv + (inc - v)], d2, mask=m)
                cv = cv + plsc.all_reduce_population_count(m)
            cntv[pl.ds(coff, 16)] = cv
            cntv[pl.ds(doff, 16)] = (cv >= cap).astype(jnp.int32)

        def chunk(jj, _):
            off4 = jj * 64
            d1 = cntv[pl.ds(32, 16)][0]
            d2f = cntv[pl.ds(48, 16)][0]

            @pl.when(d1 + d2f < 2)
            def _():
                d2s = []
                for k in range(4):
                    xs = xv[pl.ds(off4 + 16 * k, 16)]
                    ys = yv[pl.ds(off4 + 16 * k, 16)]
                    zs = zv[pl.ds(off4 + 16 * k, 16)]
                    dx = xs - cx
                    dy = ys - cy
                    dz = zs - cz
                    d2s.append(dx * dx + dy * dy + dz * dz)

                @pl.when(d1 == 0)
                def _():
                    branch(d2s, off4, t1i, t1d, 0, 32, r1sq, S1)

                @pl.when(d2f == 0)
                def _():
                    branch(d2s, off4, t2i, t2d, 16, 48, r2sq, S2)

            return 0

        lax.fori_loop(0, NSTEP // 4, chunk, 0)
        c1 = jnp.minimum(cntv[pl.ds(0, 16)][0], S1)
        c2 = jnp.minimum(cntv[pl.ds(16, 16)][0], S2)

        cc = jnp.full((16,), c, jnp.int32)
        f_i = t1i[pl.ds(0, 16)][0]
        f_d = t1d[pl.ds(0, 16)][0]
        vi = jnp.where(lane >= c1, f_i, t1i[pl.ds(0, 16)])
        vd = jnp.where(lane >= c1, f_d, t1d[pl.ds(0, 16)])
        o1i[c, :] = vi
        loc = vi - base
        gx = plsc.load_gather(xv, [loc])
        gy = plsc.load_gather(yv, [loc])
        gz = plsc.load_gather(zv, [loc])
        plsc.store_scatter(o1g, [cc, lane, jnp.full((16,), 0, jnp.int32)], gx)
        plsc.store_scatter(o1g, [cc, lane, jnp.full((16,), 1, jnp.int32)], gy)
        plsc.store_scatter(o1g, [cc, lane, jnp.full((16,), 2, jnp.int32)], gz)
        plsc.store_scatter(o1g, [cc, lane, jnp.full((16,), 3, jnp.int32)], vd)
        f_i2 = t2i[pl.ds(0, 16)][0]
        f_d2 = t2d[pl.ds(0, 16)][0]
        for k in range(2):
            lk = lane + (16 * k)
            vi = jnp.where(lk >= c2, f_i2, t2i[pl.ds(16 * k, 16)])
            vd = jnp.where(lk >= c2, f_d2, t2d[pl.ds(16 * k, 16)])
            o2i[c, pl.ds(16 * k, 16)] = vi
            loc = vi - base
            gx = plsc.load_gather(xv, [loc])
            gy = plsc.load_gather(yv, [loc])
            gz = plsc.load_gather(zv, [loc])
            plsc.store_scatter(o2g, [cc, lk, jnp.full((16,), 0, jnp.int32)], gx)
            plsc.store_scatter(o2g, [cc, lk, jnp.full((16,), 1, jnp.int32)], gy)
            plsc.store_scatter(o2g, [cc, lk, jnp.full((16,), 2, jnp.int32)], gz)
            plsc.store_scatter(o2g, [cc, lk, jnp.full((16,), 3, jnp.int32)], vd)
        return 0

    lax.fori_loop(0, CPW, centroid, 0)

    pltpu.sync_copy(o1i, i1_hbm.at[b, pl.ds(p0, CPW)])
    pltpu.sync_copy(o2i, i2_hbm.at[b, pl.ds(p0, CPW)])
    pltpu.sync_copy(o1g, g1_hbm.at[b, pl.ds(p0, CPW)])
    pltpu.sync_copy(o2g, g2_hbm.at[b, pl.ds(p0, CPW)])


def _run_ball_query(planes, cplanes):
    mesh = plsc.VectorSubcoreMesh(core_axis_name="c", subcore_axis_name="s",
                                   num_cores=2, num_subcores=16)
    f = pl.kernel(
        _bq_body,
        out_type=[
            pltpu.HBM((B, P, S1), jnp.int32),
            pltpu.HBM((B, P, S1, 4), jnp.float32),
            pltpu.HBM((B, P, S2), jnp.int32),
            pltpu.HBM((B, P, S2, 4), jnp.float32),
        ],
        mesh=mesh,
        compiler_params=pltpu.CompilerParams(needs_layout_passes=False,
                                             use_tc_tiling_on_sc=False),
        scratch_types=[
            pltpu.VMEM((N,), jnp.float32),
            pltpu.VMEM((N,), jnp.float32),
            pltpu.VMEM((N,), jnp.float32),
            pltpu.VMEM((CPW + 16,), jnp.float32),
            pltpu.VMEM((CPW + 16,), jnp.float32),
            pltpu.VMEM((CPW + 16,), jnp.float32),
            pltpu.VMEM((CPW, S1), jnp.int32),
            pltpu.VMEM((CPW, S2), jnp.int32),
            pltpu.VMEM((CPW, S1, 4), jnp.float32),
            pltpu.VMEM((CPW, S2, 4), jnp.float32),
            pltpu.VMEM((48,), jnp.int32),
            pltpu.VMEM((48,), jnp.float32),
            pltpu.VMEM((48,), jnp.int32),
            pltpu.VMEM((48,), jnp.float32),
            pltpu.VMEM((64,), jnp.int32),
            pltpu.VMEM((64,), jnp.float32),
        ],
    )
    return f(*planes, *cplanes)


def _gather_body(table_hbm, i1_hbm, i2_hbm, g1_hbm, g2_hbm,
                 idxv, buf, sem):
    w = lax.axis_index("c") * 16 + lax.axis_index("s")

    def run(idx_hbm, out_hbm, rows_per_sub):
        nchunk = rows_per_sub // 128
        r0 = w * rows_per_sub

        def chunk(c, _):
            rb = r0 + c * 128
            pltpu.sync_copy(idx_hbm.at[pl.ds(rb, 128)], idxv)
            pltpu.async_copy(table_hbm.at[idxv], buf, sem).wait()
            pltpu.sync_copy(buf, out_hbm.at[pl.ds(rb, 128)])
            return 0

        lax.fori_loop(0, nchunk, chunk, 0)

    run(i1_hbm, g1_hbm, M1 // NSUB)
    run(i2_hbm, g2_hbm, M2 // NSUB)


def _run_gather(table, idx1f, idx2f):
    mesh = plsc.VectorSubcoreMesh(core_axis_name="c", subcore_axis_name="s",
                                   num_cores=2, num_subcores=16)
    f = pl.kernel(
        _gather_body,
        out_type=[
            pltpu.HBM((M1, CF), jnp.float32),
            pltpu.HBM((M2, CF), jnp.float32),
        ],
        mesh=mesh,
        compiler_params=pltpu.CompilerParams(needs_layout_passes=False,
                                             use_tc_tiling_on_sc=False),
        scratch_types=[
            pltpu.VMEM((128,), jnp.int32),
            pltpu.VMEM((128, CF), jnp.float32),
            pltpu.SemaphoreType.DMA,
        ],
    )
    return f(table, idx1f, idx2f)


def _geom_h1(geo, cent, w1t, S, bm):
    q = bm // S
    geo3 = geo.reshape(q, S, 4)
    raw = geo3[:, :, 0:3]
    d2 = geo3[:, :, 3:4]
    delta = raw - cent.reshape(q, 1, 3)
    xi = jnp.broadcast_to(geo3[:, 0:1, 0:3], (q, S, 3))
    dist = jnp.sqrt(d2 + 1e-12)
    h10 = jnp.concatenate([dist, xi, raw, delta], axis=-1).reshape(bm, 10)
    h1 = jnp.dot(h10, w1t, preferred_element_type=jnp.float32)
    return h1, delta


def _pass1_body(geo_ref, cent_ref, w1t_ref, acc_ref, *, S, bm):
    h1, _ = _geom_h1(geo_ref[...], cent_ref[...], w1t_ref[...], S, bm)

    @pl.when(pl.program_id(0) == 0)
    def _():
        acc_ref[...] = jnp.zeros((8, 128), jnp.float32)

    acc_ref[0:1, 0:CMID] += jnp.sum(h1, axis=0, keepdims=True)
    acc_ref[1:2, 0:CMID] += jnp.sum(h1 * h1, axis=0, keepdims=True)


def _pass2_body(geo_ref, cent_ref, feat_ref, w1t_ref, acc1_ref, w2pt_ref,
                b2p_ref, m_ref, accy_ref, *, S, bm, mbr):
    q = bm // S
    h1, delta = _geom_h1(geo_ref[...], cent_ref[...], w1t_ref[...], S, bm)
    s1 = acc1_ref[0:1, 0:CMID]
    q1 = acc1_ref[1:2, 0:CMID]
    mu1 = s1 * (1.0 / mbr)
    var1 = q1 * (1.0 / mbr) - mu1 * mu1
    rs1 = lax.rsqrt(var1 + EPS)
    h = jnp.maximum((h1 - mu1) * rs1, 0.0)
    h2 = jnp.dot(h, w2pt_ref[...], preferred_element_type=jnp.float32)
    h2 = h2 + b2p_ref[...]
    x = jnp.concatenate([feat_ref[...].reshape(q, S, CF), delta], axis=-1)
    y3 = h2.reshape(q, S, CIN) * x
    y2 = y3.reshape(bm, CIN)
    m = jnp.max(y3, axis=1)
    m_ref[...] = jnp.concatenate(
        [m, jnp.zeros((q, 128 - CIN), jnp.float32)], axis=-1)

    @pl.when(pl.program_id(0) == 0)
    def _():
        accy_ref[...] = jnp.zeros((8, 128), jnp.float32)

    accy_ref[0:1, 0:CIN] += jnp.sum(y2, axis=0, keepdims=True)
    accy_ref[1:2, 0:CIN] += jnp.sum(y2 * y2, axis=0, keepdims=True)


def _pass3_body(m1_ref, a1_ref, m2_ref, a2_ref, wcrt_ref, o1_ref, o2_ref):
    for m_ref, a_ref, o_ref, mbr in ((m1_ref, a1_ref, o1_ref, M1),
                                     (m2_ref, a2_ref, o2_ref, M2)):
        sy = a_ref[0:1, 0:CIN]
        qy = a_ref[1:2, 0:CIN]
        mu = sy * (1.0 / mbr)
        var = qy * (1.0 / mbr) - mu * mu
        rs = lax.rsqrt(var + EPS)
        x3 = jnp.maximum((m_ref[...][:, 0:CIN] - mu) * rs, 0.0)
        z = jnp.dot(x3, wcrt_ref[...], preferred_element_type=jnp.float32)
        muz = jnp.sum(z, axis=0, keepdims=True) * (1.0 / (B * P))
        varz = jnp.sum(z * z, axis=0, keepdims=True) * (1.0 / (B * P)) \
            - muz * muz
        o_ref[...] = jnp.maximum((z - muz) * lax.rsqrt(varz + EPS), 0.0)


def _run_branch(geo, cent, feat, w1t, w2pt, b2p, S, mbr):
    bm = 4096
    grid = (mbr // bm,)
    q = bm // S
    acc1 = pl.pallas_call(
        functools.partial(_pass1_body, S=S, bm=bm),
        grid=grid,
        in_specs=[
            pl.BlockSpec((bm, 4), lambda i: (i, 0)),
            pl.BlockSpec((q, 3), lambda i: (i, 0)),
            pl.BlockSpec((10, CMID), lambda i: (0, 0)),
        ],
        out_specs=pl.BlockSpec((8, 128), lambda i: (0, 0)),
        out_shape=jax.ShapeDtypeStruct((8, 128), jnp.float32),
    )(geo, cent, w1t)
    m, accy = pl.pallas_call(
        functools.partial(_pass2_body, S=S, bm=bm, mbr=mbr),
        grid=grid,
        in_specs=[
            pl.BlockSpec((bm, 4), lambda i: (i, 0)),
            pl.BlockSpec((q, 3), lambda i: (i, 0)),
            pl.BlockSpec((bm, CF), lambda i: (i, 0)),
            pl.BlockSpec((10, CMID), lambda i: (0, 0)),
            pl.BlockSpec((8, 128), lambda i: (0, 0)),
            pl.BlockSpec((CMID, CIN), lambda i: (0, 0)),
            pl.BlockSpec((1, CIN), lambda i: (0, 0)),
        ],
        out_specs=[
            pl.BlockSpec((q, 128), lambda i: (i, 0)),
            pl.BlockSpec((8, 128), lambda i: (0, 0)),
        ],
        out_shape=[
            jax.ShapeDtypeStruct((B * P, 128), jnp.float32),
            jax.ShapeDtypeStruct((8, 128), jnp.float32),
        ],
    )(geo, cent, feat, w1t, acc1, w2pt, b2p)
    return m, accy


def kernel(xyz, features, w1, b1, w2, b2, wcr, bcr):
    xt = jnp.transpose(xyz, (0, 2, 1))
    o = _run_fps(xt, xyz)
    op = o.reshape(P, B, 4)
    new_xyz = jnp.transpose(op[:, :, 0:3], (1, 0, 2))

    planes = (xt[:, 0], xt[:, 1], xt[:, 2])
    cplanes = (op[:, :, 0].T, op[:, :, 1].T, op[:, :, 2].T)
    idx1, geo1, idx2, geo2 = _run_ball_query(planes, cplanes)

    table = jnp.transpose(features, (0, 2, 1)).reshape(B * N, CF)
    feat1, feat2 = _run_gather(table, idx1.reshape(M1), idx2.reshape(M2))

    perm = jnp.concatenate(
        [jnp.arange(3, CIN, dtype=jnp.int32),
         jnp.arange(0, 3, dtype=jnp.int32)])
    w1t = jnp.transpose(w1)
    w2pt = jnp.transpose(w2[perm, :])
    b2p = b2[perm].reshape(1, CIN)
    wcrt = jnp.transpose(wcr[:, perm])

    cent = new_xyz.reshape(B * P, 3)
    m1, ay1 = _run_branch(geo1.reshape(M1, 4), cent, feat1,
                          w1t, w2pt, b2p, S1, M1)
    m2, ay2 = _run_branch(geo2.reshape(M2, 4), cent, feat2,
                          w1t, w2pt, b2p, S2, M2)

    o1, o2 = pl.pallas_call(
        _pass3_body,
        out_shape=[jax.ShapeDtypeStruct((B * P, COUT), jnp.float32)] * 2,
    )(m1, ay1, m2, ay2, wcrt)

    f1 = o1.reshape(B, P, COUT).transpose(0, 2, 1)
    f2 = o2.reshape(B, P, COUT).transpose(0, 2, 1)
    return (new_xyz, jnp.concatenate([f1, f2], axis=1))


# speedup vs baseline: 1.4760x; 1.0609x over previous
import functools

import jax
import jax.numpy as jnp
from jax import lax
from jax.experimental import pallas as pl
from jax.experimental.pallas import tpu as pltpu
from jax.experimental.pallas import tpu_sc as plsc

B = 4
N = 8192
P = 1024
R1, R2 = 0.1, 0.2
S1, S2 = 16, 32
CF = 64
CIN = CF + 3
CMID = 32
COUT = 128
EPS = 1e-5
M1 = B * P * S1
M2 = B * P * S2
NSUB = 32
CPW = P // (NSUB // B)
NSTEP = N // 16


def _fps_body(x_ref, y_ref, z_ref, ox_ref, oy_ref, oz_ref, dists_ref):
    gidx = (lax.broadcasted_iota(jnp.int32, (B, 8, N // 8), 1) * (N // 8)
            + lax.broadcasted_iota(jnp.int32, (B, 8, N // 8), 2))

    dists_ref[...] = jnp.full((B, 8, N // 8), 1e10, jnp.float32)
    zf = jnp.zeros((B, 8, N // 8), jnp.float32)

    def extract(iv):
        m = gidx == iv
        lx = jnp.sum(jnp.sum(jnp.where(m, x_ref[...], zf), axis=2,
                             keepdims=True), axis=1, keepdims=True)
        ly = jnp.sum(jnp.sum(jnp.where(m, y_ref[...], zf), axis=2,
                             keepdims=True), axis=1, keepdims=True)
        lz = jnp.sum(jnp.sum(jnp.where(m, z_ref[...], zf), axis=2,
                             keepdims=True), axis=1, keepdims=True)
        return lx, ly, lz

    def store(i, lx, ly, lz):
        ox_ref[:, pl.ds(i, 1), :] = lx
        oy_ref[:, pl.ds(i, 1), :] = ly
        oz_ref[:, pl.ds(i, 1), :] = lz

    lx, ly, lz = extract(jnp.zeros((B, 1, 1), jnp.int32))
    store(0, lx, ly, lz)

    def body(i, carry):
        lx, ly, lz = carry
        dx = x_ref[...] - lx
        dy = y_ref[...] - ly
        dz = z_ref[...] - lz
        d2 = dx * dx + dy * dy + dz * dz
        dm = jnp.minimum(dists_ref[...], d2)
        dists_ref[...] = dm
        mx = jnp.max(jnp.max(dm, axis=2, keepdims=True), axis=1,
                     keepdims=True)
        cand = jnp.where(dm == mx, gidx, N)
        iv = jnp.min(jnp.min(cand, axis=2, keepdims=True), axis=1,
                     keepdims=True)
        lx, ly, lz = extract(iv)
        store(i, lx, ly, lz)
        return (lx, ly, lz)

    lax.fori_loop(1, P, body, (lx, ly, lz))


def _run_fps(xt):
    x = xt[:, 0, :].reshape(B, 8, N // 8)
    y = xt[:, 1, :].reshape(B, 8, N // 8)
    z = xt[:, 2, :].reshape(B, 8, N // 8)
    out = pl.pallas_call(
        _fps_body,
        out_shape=[jax.ShapeDtypeStruct((B, P, 1), jnp.float32)] * 3,
        scratch_shapes=[pltpu.VMEM((B, 8, N // 8), jnp.float32)],
    )(x, y, z)
    return out


def _bq_body(xp_hbm, yp_hbm, zp_hbm, cxp_hbm, cyp_hbm, czp_hbm,
             i1_hbm, g1_hbm, i2_hbm, g2_hbm,
             xv, yv, zv, cxv, cyv, czv, o1i, o2i, o1g, o2g,
             t1i, t1d, t2i, t2d, cntv, d2b):
    w = lax.axis_index("c") * 16 + lax.axis_index("s")
    b = w // (NSUB // B)
    p0 = (w % (NSUB // B)) * CPW
    base = b * N

    pltpu.sync_copy(xp_hbm.at[b], xv)
    pltpu.sync_copy(yp_hbm.at[b], yv)
    pltpu.sync_copy(zp_hbm.at[b], zv)
    pltpu.sync_copy(cxp_hbm.at[b, pl.ds(p0, CPW)], cxv.at[pl.ds(0, CPW)])
    pltpu.sync_copy(cyp_hbm.at[b, pl.ds(p0, CPW)], cyv.at[pl.ds(0, CPW)])
    pltpu.sync_copy(czp_hbm.at[b, pl.ds(p0, CPW)], czv.at[pl.ds(0, CPW)])

    lane = lax.broadcasted_iota(jnp.int32, (16,), 0)
    r1sq = jnp.float32(R1 * R1)
    r2sq = jnp.float32(R2 * R2)

    def centroid(c, _):
        cx = cxv[pl.ds(c, 16)][0]
        cy = cyv[pl.ds(c, 16)][0]
        cz = czv[pl.ds(c, 16)][0]
        x0 = xv[pl.ds(0, 16)][0]
        y0 = yv[pl.ds(0, 16)][0]
        z0 = zv[pl.ds(0, 16)][0]
        d20 = ((x0 - cx) * (x0 - cx) + (y0 - cy) * (y0 - cy)
               + (z0 - cz) * (z0 - cz))
        t1i[pl.ds(0, 16)] = jnp.full((16,), base, jnp.int32)
        t1d[pl.ds(0, 16)] = jnp.full((16,), d20, jnp.float32)
        t2i[pl.ds(0, 16)] = jnp.full((16,), base, jnp.int32)
        t2d[pl.ds(0, 16)] = jnp.full((16,), d20, jnp.float32)

        zero16 = jnp.zeros((16,), jnp.int32)
        cntv[pl.ds(0, 16)] = zero16
        cntv[pl.ds(16, 16)] = zero16
        cntv[pl.ds(32, 16)] = zero16
        cntv[pl.ds(48, 16)] = zero16

        def branch(d2s, off4, tbi, tbd, coff, doff, rsq, cap):
            cv = cntv[pl.ds(coff, 16)]
            for k in range(4):
                d2 = d2s[k]
                gi = (base + off4 + 16 * k) + lane
                m = (d2 < rsq) & (cv < cap)
                v = m.astype(jnp.int32)
                inc = plsc.cumsum(v)
                plsc.store_scatter(tbi, [cv + (inc - v)], gi, mask=m)
                plsc.store_scatter(tbd, [cv + (inc - v)], d2, mask=m)
                cv = cv + plsc.all_reduce_population_count(m)
            cntv[pl.ds(coff, 16)] = cv
            cntv[pl.ds(doff, 16)] = (cv >= cap).astype(jnp.int32)

        def chunk(jj, _):
            off4 = jj * 64
            d1 = cntv[pl.ds(32, 16)][0]
            d2f = cntv[pl.ds(48, 16)][0]

            @pl.when(d1 + d2f < 2)
            def _():
                d2s = []
                for k in range(4):
                    xs = xv[pl.ds(off4 + 16 * k, 16)]
                    ys = yv[pl.ds(off4 + 16 * k, 16)]
                    zs = zv[pl.ds(off4 + 16 * k, 16)]
                    dx = xs - cx
                    dy = ys - cy
                    dz = zs - cz
                    d2s.append(dx * dx + dy * dy + dz * dz)

                @pl.when(d1 == 0)
                def _():
                    branch(d2s, off4, t1i, t1d, 0, 32, r1sq, S1)

                @pl.when(d2f == 0)
                def _():
                    branch(d2s, off4, t2i, t2d, 16, 48, r2sq, S2)

            return 0

        lax.fori_loop(0, NSTEP // 4, chunk, 0)
        c1 = jnp.minimum(cntv[pl.ds(0, 16)][0], S1)
        c2 = jnp.minimum(cntv[pl.ds(16, 16)][0], S2)

        cc = jnp.full((16,), c, jnp.int32)
        f_i = t1i[pl.ds(0, 16)][0]
        f_d = t1d[pl.ds(0, 16)][0]
        vi = jnp.where(lane >= c1, f_i, t1i[pl.ds(0, 16)])
        vd = jnp.where(lane >= c1, f_d, t1d[pl.ds(0, 16)])
        o1i[c, :] = vi
        loc = vi - base
        gx = plsc.load_gather(xv, [loc])
        gy = plsc.load_gather(yv, [loc])
        gz = plsc.load_gather(zv, [loc])
        plsc.store_scatter(o1g, [cc, lane, jnp.full((16,), 0, jnp.int32)], gx)
        plsc.store_scatter(o1g, [cc, lane, jnp.full((16,), 1, jnp.int32)], gy)
        plsc.store_scatter(o1g, [cc, lane, jnp.full((16,), 2, jnp.int32)], gz)
        plsc.store_scatter(o1g, [cc, lane, jnp.full((16,), 3, jnp.int32)], vd)
        f_i2 = t2i[pl.ds(0, 16)][0]
        f_d2 = t2d[pl.ds(0, 16)][0]
        for k in range(2):
            lk = lane + (16 * k)
            vi = jnp.where(lk >= c2, f_i2, t2i[pl.ds(16 * k, 16)])
            vd = jnp.where(lk >= c2, f_d2, t2d[pl.ds(16 * k, 16)])
            o2i[c, pl.ds(16 * k, 16)] = vi
            loc = vi - base
            gx = plsc.load_gather(xv, [loc])
            gy = plsc.load_gather(yv, [loc])
            gz = plsc.load_gather(zv, [loc])
            plsc.store_scatter(o2g, [cc, lk, jnp.full((16,), 0, jnp.int32)], gx)
            plsc.store_scatter(o2g, [cc, lk, jnp.full((16,), 1, jnp.int32)], gy)
            plsc.store_scatter(o2g, [cc, lk, jnp.full((16,), 2, jnp.int32)], gz)
            plsc.store_scatter(o2g, [cc, lk, jnp.full((16,), 3, jnp.int32)], vd)
        return 0

    lax.fori_loop(0, CPW, centroid, 0)

    pltpu.sync_copy(o1i, i1_hbm.at[b, pl.ds(p0, CPW)])
    pltpu.sync_copy(o2i, i2_hbm.at[b, pl.ds(p0, CPW)])
    pltpu.sync_copy(o1g, g1_hbm.at[b, pl.ds(p0, CPW)])
    pltpu.sync_copy(o2g, g2_hbm.at[b, pl.ds(p0, CPW)])


def _run_ball_query(planes, cplanes):
    mesh = plsc.VectorSubcoreMesh(core_axis_name="c", subcore_axis_name="s",
                                   num_cores=2, num_subcores=16)
    f = pl.kernel(
        _bq_body,
        out_type=[
            pltpu.HBM((B, P, S1), jnp.int32),
            pltpu.HBM((B, P, S1, 4), jnp.float32),
            pltpu.HBM((B, P, S2), jnp.int32),
            pltpu.HBM((B, P, S2, 4), jnp.float32),
        ],
        mesh=mesh,
        compiler_params=pltpu.CompilerParams(needs_layout_passes=False,
                                             use_tc_tiling_on_sc=False),
        scratch_types=[
            pltpu.VMEM((N,), jnp.float32),
            pltpu.VMEM((N,), jnp.float32),
            pltpu.VMEM((N,), jnp.float32),
            pltpu.VMEM((CPW + 16,), jnp.float32),
            pltpu.VMEM((CPW + 16,), jnp.float32),
            pltpu.VMEM((CPW + 16,), jnp.float32),
            pltpu.VMEM((CPW, S1), jnp.int32),
            pltpu.VMEM((CPW, S2), jnp.int32),
            pltpu.VMEM((CPW, S1, 4), jnp.float32),
            pltpu.VMEM((CPW, S2, 4), jnp.float32),
            pltpu.VMEM((48,), jnp.int32),
            pltpu.VMEM((48,), jnp.float32),
            pltpu.VMEM((48,), jnp.int32),
            pltpu.VMEM((48,), jnp.float32),
            pltpu.VMEM((64,), jnp.int32),
            pltpu.VMEM((64,), jnp.float32),
        ],
    )
    return f(*planes, *cplanes)


def _gather_body(table_hbm, i1_hbm, i2_hbm, g1_hbm, g2_hbm,
                 idxv, buf, sem):
    w = lax.axis_index("c") * 16 + lax.axis_index("s")

    def run(idx_hbm, out_hbm, rows_per_sub):
        nchunk = rows_per_sub // 128
        r0 = w * rows_per_sub

        def chunk(c, _):
            rb = r0 + c * 128
            pltpu.sync_copy(idx_hbm.at[pl.ds(rb, 128)], idxv)
            pltpu.async_copy(table_hbm.at[idxv], buf, sem).wait()
            pltpu.sync_copy(buf, out_hbm.at[pl.ds(rb, 128)])
            return 0

        lax.fori_loop(0, nchunk, chunk, 0)

    run(i1_hbm, g1_hbm, M1 // NSUB)
    run(i2_hbm, g2_hbm, M2 // NSUB)


def _run_gather(table, idx1f, idx2f):
    mesh = plsc.VectorSubcoreMesh(core_axis_name="c", subcore_axis_name="s",
                                   num_cores=2, num_subcores=16)
    f = pl.kernel(
        _gather_body,
        out_type=[
            pltpu.HBM((M1, CF), jnp.float32),
            pltpu.HBM((M2, CF), jnp.float32),
        ],
        mesh=mesh,
        compiler_params=pltpu.CompilerParams(needs_layout_passes=False,
                                             use_tc_tiling_on_sc=False),
        scratch_types=[
            pltpu.VMEM((128,), jnp.int32),
            pltpu.VMEM((128, CF), jnp.float32),
            pltpu.SemaphoreType.DMA,
        ],
    )
    return f(table, idx1f, idx2f)


def _geom_h1(geo, cent, w1t, S, bm):
    q = bm // S
    geo3 = geo.reshape(q, S, 4)
    raw = geo3[:, :, 0:3]
    d2 = geo3[:, :, 3:4]
    delta = raw - cent.reshape(q, 1, 3)
    xi = jnp.broadcast_to(geo3[:, 0:1, 0:3], (q, S, 3))
    dist = jnp.sqrt(d2 + 1e-12)
    h10 = jnp.concatenate([dist, xi, raw, delta], axis=-1).reshape(bm, 10)
    h1 = jnp.dot(h10, w1t, preferred_element_type=jnp.float32)
    return h1, delta


def _pass1_body(geo_ref, cent_ref, w1t_ref, acc_ref, *, S, bm):
    h1, _ = _geom_h1(geo_ref[...], cent_ref[...], w1t_ref[...], S, bm)

    @pl.when(pl.program_id(0) == 0)
    def _():
        acc_ref[...] = jnp.zeros((8, 128), jnp.float32)

    acc_ref[0:1, 0:CMID] += jnp.sum(h1, axis=0, keepdims=True)
    acc_ref[1:2, 0:CMID] += jnp.sum(h1 * h1, axis=0, keepdims=True)


def _pass2_body(geo_ref, cent_ref, feat_ref, w1t_ref, acc1_ref, w2pt_ref,
                b2p_ref, m_ref, accy_ref, *, S, bm, mbr):
    q = bm // S
    h1, delta = _geom_h1(geo_ref[...], cent_ref[...], w1t_ref[...], S, bm)
    s1 = acc1_ref[0:1, 0:CMID]
    q1 = acc1_ref[1:2, 0:CMID]
    mu1 = s1 * (1.0 / mbr)
    var1 = q1 * (1.0 / mbr) - mu1 * mu1
    rs1 = lax.rsqrt(var1 + EPS)
    h = jnp.maximum((h1 - mu1) * rs1, 0.0)
    h2 = jnp.dot(h, w2pt_ref[...], preferred_element_type=jnp.float32)
    h2 = h2 + b2p_ref[...]
    x = jnp.concatenate([feat_ref[...].reshape(q, S, CF), delta], axis=-1)
    y3 = h2.reshape(q, S, CIN) * x
    y2 = y3.reshape(bm, CIN)
    m = jnp.max(y3, axis=1)
    m_ref[...] = jnp.concatenate(
        [m, jnp.zeros((q, 128 - CIN), jnp.float32)], axis=-1)

    @pl.when(pl.program_id(0) == 0)
    def _():
        accy_ref[...] = jnp.zeros((8, 128), jnp.float32)

    accy_ref[0:1, 0:CIN] += jnp.sum(y2, axis=0, keepdims=True)
    accy_ref[1:2, 0:CIN] += jnp.sum(y2 * y2, axis=0, keepdims=True)


def _pass3_body(m1_ref, a1_ref, m2_ref, a2_ref, wcrt_ref, o1_ref, o2_ref):
    for m_ref, a_ref, o_ref, mbr in ((m1_ref, a1_ref, o1_ref, M1),
                                     (m2_ref, a2_ref, o2_ref, M2)):
        sy = a_ref[0:1, 0:CIN]
        qy = a_ref[1:2, 0:CIN]
        mu = sy * (1.0 / mbr)
        var = qy * (1.0 / mbr) - mu * mu
        rs = lax.rsqrt(var + EPS)
        x3 = jnp.maximum((m_ref[...][:, 0:CIN] - mu) * rs, 0.0)
        z = jnp.dot(x3, wcrt_ref[...], preferred_element_type=jnp.float32)
        muz = jnp.sum(z, axis=0, keepdims=True) * (1.0 / (B * P))
        varz = jnp.sum(z * z, axis=0, keepdims=True) * (1.0 / (B * P)) \
            - muz * muz
        o_ref[...] = jnp.maximum((z - muz) * lax.rsqrt(varz + EPS), 0.0)


def _run_branch(geo, cent, feat, w1t, w2pt, b2p, S, mbr):
    bm = 4096
    grid = (mbr // bm,)
    q = bm // S
    acc1 = pl.pallas_call(
        functools.partial(_pass1_body, S=S, bm=bm),
        grid=grid,
        in_specs=[
            pl.BlockSpec((bm, 4), lambda i: (i, 0)),
            pl.BlockSpec((q, 3), lambda i: (i, 0)),
            pl.BlockSpec((10, CMID), lambda i: (0, 0)),
        ],
        out_specs=pl.BlockSpec((8, 128), lambda i: (0, 0)),
        out_shape=jax.ShapeDtypeStruct((8, 128), jnp.float32),
    )(geo, cent, w1t)
    m, accy = pl.pallas_call(
        functools.partial(_pass2_body, S=S, bm=bm, mbr=mbr),
        grid=grid,
        in_specs=[
            pl.BlockSpec((bm, 4), lambda i: (i, 0)),
            pl.BlockSpec((q, 3), lambda i: (i, 0)),
            pl.BlockSpec((bm, CF), lambda i: (i, 0)),
            pl.BlockSpec((10, CMID), lambda i: (0, 0)),
            pl.BlockSpec((8, 128), lambda i: (0, 0)),
            pl.BlockSpec((CMID, CIN), lambda i: (0, 0)),
            pl.BlockSpec((1, CIN), lambda i: (0, 0)),
        ],
        out_specs=[
            pl.BlockSpec((q, 128), lambda i: (i, 0)),
            pl.BlockSpec((8, 128), lambda i: (0, 0)),
        ],
        out_shape=[
            jax.ShapeDtypeStruct((B * P, 128), jnp.float32),
            jax.ShapeDtypeStruct((8, 128), jnp.float32),
        ],
    )(geo, cent, feat, w1t, acc1, w2pt, b2p)
    return m, accy


def kernel(xyz, features, w1, b1, w2, b2, wcr, bcr):
    xt = jnp.transpose(xyz, (0, 2, 1))
    ox3, oy3, oz3 = _run_fps(xt)
    new_xyz = jnp.concatenate([ox3, oy3, oz3], axis=-1)

    planes = (xt[:, 0], xt[:, 1], xt[:, 2])
    cplanes = (ox3[..., 0], oy3[..., 0], oz3[..., 0])
    idx1, geo1, idx2, geo2 = _run_ball_query(planes, cplanes)

    table = jnp.transpose(features, (0, 2, 1)).reshape(B * N, CF)
    feat1, feat2 = _run_gather(table, idx1.reshape(M1), idx2.reshape(M2))

    perm = jnp.concatenate(
        [jnp.arange(3, CIN, dtype=jnp.int32),
         jnp.arange(0, 3, dtype=jnp.int32)])
    w1t = jnp.transpose(w1)
    w2pt = jnp.transpose(w2[perm, :])
    b2p = b2[perm].reshape(1, CIN)
    wcrt = jnp.transpose(wcr[:, perm])

    cent = new_xyz.reshape(B * P, 3)
    m1, ay1 = _run_branch(geo1.reshape(M1, 4), cent, feat1,
                          w1t, w2pt, b2p, S1, M1)
    m2, ay2 = _run_branch(geo2.reshape(M2, 4), cent, feat2,
                          w1t, w2pt, b2p, S2, M2)

    o1, o2 = pl.pallas_call(
        _pass3_body,
        out_shape=[jax.ShapeDtypeStruct((B * P, COUT), jnp.float32)] * 2,
    )(m1, ay1, m2, ay2, wcrt)

    f1 = o1.reshape(B, P, COUT).transpose(0, 2, 1)
    f2 = o2.reshape(B, P, COUT).transpose(0, 2, 1)
    return (new_xyz, jnp.concatenate([f1, f2], axis=1))


# speedup vs baseline: 1.8652x; 1.2636x over previous
import functools

import jax
import jax.numpy as jnp
from jax import lax
from jax.experimental import pallas as pl
from jax.experimental.pallas import tpu as pltpu
from jax.experimental.pallas import tpu_sc as plsc

B = 4
N = 8192
P = 1024
R1, R2 = 0.1, 0.2
S1, S2 = 16, 32
CF = 64
CIN = CF + 3
CMID = 32
COUT = 128
EPS = 1e-5
M1 = B * P * S1
M2 = B * P * S2
NSUB = 32
CPW = P // (NSUB // B)
NSTEP = N // 16


def _fps_body(x_ref, y_ref, z_ref, ox_ref, oy_ref, oz_ref, dists_ref):
    gidx = (lax.broadcasted_iota(jnp.int32, (B, 8, N // 8), 1) * (N // 8)
            + lax.broadcasted_iota(jnp.int32, (B, 8, N // 8), 2))

    dists_ref[...] = jnp.full((B, 8, N // 8), 1e10, jnp.float32)
    zf = jnp.zeros((B, 8, N // 8), jnp.float32)

    def extract(iv):
        m = gidx == iv
        lx = jnp.sum(jnp.sum(jnp.where(m, x_ref[...], zf), axis=2,
                             keepdims=True), axis=1, keepdims=True)
        ly = jnp.sum(jnp.sum(jnp.where(m, y_ref[...], zf), axis=2,
                             keepdims=True), axis=1, keepdims=True)
        lz = jnp.sum(jnp.sum(jnp.where(m, z_ref[...], zf), axis=2,
                             keepdims=True), axis=1, keepdims=True)
        return lx, ly, lz

    def store(i, lx, ly, lz):
        ox_ref[:, pl.ds(i, 1), :] = lx
        oy_ref[:, pl.ds(i, 1), :] = ly
        oz_ref[:, pl.ds(i, 1), :] = lz

    lx, ly, lz = extract(jnp.zeros((B, 1, 1), jnp.int32))
    store(0, lx, ly, lz)

    def body(i, carry):
        lx, ly, lz = carry
        dx = x_ref[...] - lx
        dy = y_ref[...] - ly
        dz = z_ref[...] - lz
        d2 = dx * dx + dy * dy + dz * dz
        dm = jnp.minimum(dists_ref[...], d2)
        dists_ref[...] = dm
        mx = jnp.max(jnp.max(dm, axis=2, keepdims=True), axis=1,
                     keepdims=True)
        cand = jnp.where(dm == mx, gidx, N)
        iv = jnp.min(jnp.min(cand, axis=2, keepdims=True), axis=1,
                     keepdims=True)
        lx, ly, lz = extract(iv)
        store(i, lx, ly, lz)
        return (lx, ly, lz)

    lax.fori_loop(1, P, body, (lx, ly, lz))


def _run_fps(xt):
    x = xt[:, 0, :].reshape(B, 8, N // 8)
    y = xt[:, 1, :].reshape(B, 8, N // 8)
    z = xt[:, 2, :].reshape(B, 8, N // 8)
    out = pl.pallas_call(
        _fps_body,
        out_shape=[jax.ShapeDtypeStruct((B, P, 1), jnp.float32)] * 3,
        scratch_shapes=[pltpu.VMEM((B, 8, N // 8), jnp.float32)],
    )(x, y, z)
    return out


def _bq_body(xp_hbm, yp_hbm, zp_hbm, cxp_hbm, cyp_hbm, czp_hbm,
             i1_hbm, g1_hbm, i2_hbm, g2_hbm,
             xv, yv, zv, cxv, cyv, czv, o1i, o2i, o1g, o2g,
             t1i, t1d, t2i, t2d, cntv, d2b):
    w = lax.axis_index("c") * 16 + lax.axis_index("s")
    b = w // (NSUB // B)
    p0 = (w % (NSUB // B)) * CPW
    base = b * N

    pltpu.sync_copy(xp_hbm.at[b], xv)
    pltpu.sync_copy(yp_hbm.at[b], yv)
    pltpu.sync_copy(zp_hbm.at[b], zv)
    pltpu.sync_copy(cxp_hbm.at[b, pl.ds(p0, CPW)], cxv.at[pl.ds(0, CPW)])
    pltpu.sync_copy(cyp_hbm.at[b, pl.ds(p0, CPW)], cyv.at[pl.ds(0, CPW)])
    pltpu.sync_copy(czp_hbm.at[b, pl.ds(p0, CPW)], czv.at[pl.ds(0, CPW)])

    lane = lax.broadcasted_iota(jnp.int32, (16,), 0)
    r1sq = jnp.float32(R1 * R1)
    r2sq = jnp.float32(R2 * R2)

    def centroid(c, _):
        cx = cxv[pl.ds(c, 16)][0]
        cy = cyv[pl.ds(c, 16)][0]
        cz = czv[pl.ds(c, 16)][0]
        x0 = xv[pl.ds(0, 16)][0]
        y0 = yv[pl.ds(0, 16)][0]
        z0 = zv[pl.ds(0, 16)][0]
        d20 = ((x0 - cx) * (x0 - cx) + (y0 - cy) * (y0 - cy)
               + (z0 - cz) * (z0 - cz))
        t1i[pl.ds(0, 16)] = jnp.full((16,), base, jnp.int32)
        t1d[pl.ds(0, 16)] = jnp.full((16,), d20, jnp.float32)
        t2i[pl.ds(0, 16)] = jnp.full((16,), base, jnp.int32)
        t2d[pl.ds(0, 16)] = jnp.full((16,), d20, jnp.float32)

        zero16 = jnp.zeros((16,), jnp.int32)
        cntv[pl.ds(0, 16)] = zero16
        cntv[pl.ds(16, 16)] = zero16
        cntv[pl.ds(32, 16)] = zero16
        cntv[pl.ds(48, 16)] = zero16

        def branch(d2s, off4, tbi, tbd, coff, doff, rsq, cap):
            cv = cntv[pl.ds(coff, 16)]
            for k in range(8):
                d2 = d2s[k]
                gi = (base + off4 + 16 * k) + lane
                m = (d2 < rsq) & (cv < cap)
                v = m.astype(jnp.int32)
                inc = plsc.cumsum(v)
                plsc.store_scatter(tbi, [cv + (inc - v)], gi, mask=m)
                plsc.store_scatter(tbd, [cv + (inc - v)], d2, mask=m)
                cv = cv + plsc.all_reduce_population_count(m)
            cntv[pl.ds(coff, 16)] = cv
            cntv[pl.ds(doff, 16)] = (cv >= cap).astype(jnp.int32)

        def chunk(jj, _):
            off4 = jj * 128
            d1 = cntv[pl.ds(32, 16)][0]
            d2f = cntv[pl.ds(48, 16)][0]

            @pl.when(d1 + d2f < 2)
            def _():
                d2s = []
                for k in range(8):
                    xs = xv[pl.ds(off4 + 16 * k, 16)]
                    ys = yv[pl.ds(off4 + 16 * k, 16)]
                    zs = zv[pl.ds(off4 + 16 * k, 16)]
                    dx = xs - cx
                    dy = ys - cy
                    dz = zs - cz
                    d2s.append(dx * dx + dy * dy + dz * dz)

                @pl.when(d1 == 0)
                def _():
                    branch(d2s, off4, t1i, t1d, 0, 32, r1sq, S1)

                @pl.when(d2f == 0)
                def _():
                    branch(d2s, off4, t2i, t2d, 16, 48, r2sq, S2)

            return 0

        lax.fori_loop(0, NSTEP // 8, chunk, 0)
        c1 = jnp.minimum(cntv[pl.ds(0, 16)][0], S1)
        c2 = jnp.minimum(cntv[pl.ds(16, 16)][0], S2)

        cc = jnp.full((16,), c, jnp.int32)
        f_i = t1i[pl.ds(0, 16)][0]
        f_d = t1d[pl.ds(0, 16)][0]
        vi = jnp.where(lane >= c1, f_i, t1i[pl.ds(0, 16)])
        vd = jnp.where(lane >= c1, f_d, t1d[pl.ds(0, 16)])
        o1i[c, :] = vi
        loc = vi - base
        gx = plsc.load_gather(xv, [loc])
        gy = plsc.load_gather(yv, [loc])
        gz = plsc.load_gather(zv, [loc])
        plsc.store_scatter(o1g, [cc, lane, jnp.full((16,), 0, jnp.int32)], gx)
        plsc.store_scatter(o1g, [cc, lane, jnp.full((16,), 1, jnp.int32)], gy)
        plsc.store_scatter(o1g, [cc, lane, jnp.full((16,), 2, jnp.int32)], gz)
        plsc.store_scatter(o1g, [cc, lane, jnp.full((16,), 3, jnp.int32)], vd)
        f_i2 = t2i[pl.ds(0, 16)][0]
        f_d2 = t2d[pl.ds(0, 16)][0]
        for k in range(2):
            lk = lane + (16 * k)
            vi = jnp.where(lk >= c2, f_i2, t2i[pl.ds(16 * k, 16)])
            vd = jnp.where(lk >= c2, f_d2, t2d[pl.ds(16 * k, 16)])
            o2i[c, pl.ds(16 * k, 16)] = vi
            loc = vi - base
            gx = plsc.load_gather(xv, [loc])
            gy = plsc.load_gather(yv, [loc])
            gz = plsc.load_gather(zv, [loc])
            plsc.store_scatter(o2g, [cc, lk, jnp.full((16,), 0, jnp.int32)], gx)
            plsc.store_scatter(o2g, [cc, lk, jnp.full((16,), 1, jnp.int32)], gy)
            plsc.store_scatter(o2g, [cc, lk, jnp.full((16,), 2, jnp.int32)], gz)
            plsc.store_scatter(o2g, [cc, lk, jnp.full((16,), 3, jnp.int32)], vd)
        return 0

    lax.fori_loop(0, CPW, centroid, 0)

    pltpu.sync_copy(o1i, i1_hbm.at[b, pl.ds(p0, CPW)])
    pltpu.sync_copy(o2i, i2_hbm.at[b, pl.ds(p0, CPW)])
    pltpu.sync_copy(o1g, g1_hbm.at[b, pl.ds(p0, CPW)])
    pltpu.sync_copy(o2g, g2_hbm.at[b, pl.ds(p0, CPW)])


def _run_ball_query(planes, cplanes):
    mesh = plsc.VectorSubcoreMesh(core_axis_name="c", subcore_axis_name="s",
                                   num_cores=2, num_subcores=16)
    f = pl.kernel(
        _bq_body,
        out_type=[
            pltpu.HBM((B, P, S1), jnp.int32),
            pltpu.HBM((B, P, S1, 4), jnp.float32),
            pltpu.HBM((B, P, S2), jnp.int32),
            pltpu.HBM((B, P, S2, 4), jnp.float32),
        ],
        mesh=mesh,
        compiler_params=pltpu.CompilerParams(needs_layout_passes=False,
                                             use_tc_tiling_on_sc=False),
        scratch_types=[
            pltpu.VMEM((N,), jnp.float32),
            pltpu.VMEM((N,), jnp.float32),
            pltpu.VMEM((N,), jnp.float32),
            pltpu.VMEM((CPW + 16,), jnp.float32),
            pltpu.VMEM((CPW + 16,), jnp.float32),
            pltpu.VMEM((CPW + 16,), jnp.float32),
            pltpu.VMEM((CPW, S1), jnp.int32),
            pltpu.VMEM((CPW, S2), jnp.int32),
            pltpu.VMEM((CPW, S1, 4), jnp.float32),
            pltpu.VMEM((CPW, S2, 4), jnp.float32),
            pltpu.VMEM((48,), jnp.int32),
            pltpu.VMEM((48,), jnp.float32),
            pltpu.VMEM((48,), jnp.int32),
            pltpu.VMEM((48,), jnp.float32),
            pltpu.VMEM((64,), jnp.int32),
            pltpu.VMEM((64,), jnp.float32),
        ],
    )
    return f(*planes, *cplanes)


def _gather_body(table_hbm, i1_hbm, i2_hbm, g1_hbm, g2_hbm,
                 idxv, buf, sem):
    w = lax.axis_index("c") * 16 + lax.axis_index("s")

    def run(idx_hbm, out_hbm, rows_per_sub):
        nchunk = rows_per_sub // 128
        r0 = w * rows_per_sub

        def chunk(c, _):
            rb = r0 + c * 128
            pltpu.sync_copy(idx_hbm.at[pl.ds(rb, 128)], idxv)
            pltpu.async_copy(table_hbm.at[idxv], buf, sem).wait()
            pltpu.sync_copy(buf, out_hbm.at[pl.ds(rb, 128)])
            return 0

        lax.fori_loop(0, nchunk, chunk, 0)

    run(i1_hbm, g1_hbm, M1 // NSUB)
    run(i2_hbm, g2_hbm, M2 // NSUB)


def _run_gather(table, idx1f, idx2f):
    mesh = plsc.VectorSubcoreMesh(core_axis_name="c", subcore_axis_name="s",
                                   num_cores=2, num_subcores=16)
    f = pl.kernel(
        _gather_body,
        out_type=[
            pltpu.HBM((M1, CF), jnp.float32),
            pltpu.HBM((M2, CF), jnp.float32),
        ],
        mesh=mesh,
        compiler_params=pltpu.CompilerParams(needs_layout_passes=False,
                                             use_tc_tiling_on_sc=False),
        scratch_types=[
            pltpu.VMEM((128,), jnp.int32),
            pltpu.VMEM((128, CF), jnp.float32),
            pltpu.SemaphoreType.DMA,
        ],
    )
    return f(table, idx1f, idx2f)


def _geom_h1(geo, cent, w1t, S, bm):
    q = bm // S
    geo3 = geo.reshape(q, S, 4)
    raw = geo3[:, :, 0:3]
    d2 = geo3[:, :, 3:4]
    delta = raw - cent.reshape(q, 1, 3)
    xi = jnp.broadcast_to(geo3[:, 0:1, 0:3], (q, S, 3))
    dist = jnp.sqrt(d2 + 1e-12)
    h10 = jnp.concatenate([dist, xi, raw, delta], axis=-1).reshape(bm, 10)
    h1 = jnp.dot(h10, w1t, preferred_element_type=jnp.float32)
    return h1, delta


def _pass1_body(geo_ref, cent_ref, w1t_ref, acc_ref, *, S, bm):
    h1, _ = _geom_h1(geo_ref[...], cent_ref[...], w1t_ref[...], S, bm)

    @pl.when(pl.program_id(0) == 0)
    def _():
        acc_ref[...] = jnp.zeros((8, 128), jnp.float32)

    acc_ref[0:1, 0:CMID] += jnp.sum(h1, axis=0, keepdims=True)
    acc_ref[1:2, 0:CMID] += jnp.sum(h1 * h1, axis=0, keepdims=True)


def _pass2_body(geo_ref, cent_ref, feat_ref, w1t_ref, acc1_ref, w2pt_ref,
                b2p_ref, m_ref, accy_ref, *, S, bm, mbr):
    q = bm // S
    h1, delta = _geom_h1(geo_ref[...], cent_ref[...], w1t_ref[...], S, bm)
    s1 = acc1_ref[0:1, 0:CMID]
    q1 = acc1_ref[1:2, 0:CMID]
    mu1 = s1 * (1.0 / mbr)
    var1 = q1 * (1.0 / mbr) - mu1 * mu1
    rs1 = lax.rsqrt(var1 + EPS)
    h = jnp.maximum((h1 - mu1) * rs1, 0.0)
    h2 = jnp.dot(h, w2pt_ref[...], preferred_element_type=jnp.float32)
    h2 = h2 + b2p_ref[...]
    x = jnp.concatenate([feat_ref[...].reshape(q, S, CF), delta], axis=-1)
    y3 = h2.reshape(q, S, CIN) * x
    y2 = y3.reshape(bm, CIN)
    m = jnp.max(y3, axis=1)
    m_ref[...] = jnp.concatenate(
        [m, jnp.zeros((q, 128 - CIN), jnp.float32)], axis=-1)

    @pl.when(pl.program_id(0) == 0)
    def _():
        accy_ref[...] = jnp.zeros((8, 128), jnp.float32)

    accy_ref[0:1, 0:CIN] += jnp.sum(y2, axis=0, keepdims=True)
    accy_ref[1:2, 0:CIN] += jnp.sum(y2 * y2, axis=0, keepdims=True)


def _pass3_body(m1_ref, a1_ref, m2_ref, a2_ref, wcrt_ref, o1_ref, o2_ref):
    for m_ref, a_ref, o_ref, mbr in ((m1_ref, a1_ref, o1_ref, M1),
                                     (m2_ref, a2_ref, o2_ref, M2)):
        sy = a_ref[0:1, 0:CIN]
        qy = a_ref[1:2, 0:CIN]
        mu = sy * (1.0 / mbr)
        var = qy * (1.0 / mbr) - mu * mu
        rs = lax.rsqrt(var + EPS)
        x3 = jnp.maximum((m_ref[...][:, 0:CIN] - mu) * rs, 0.0)
        z = jnp.dot(x3, wcrt_ref[...], preferred_element_type=jnp.float32)
        muz = jnp.sum(z, axis=0, keepdims=True) * (1.0 / (B * P))
        varz = jnp.sum(z * z, axis=0, keepdims=True) * (1.0 / (B * P)) \
            - muz * muz
        o_ref[...] = jnp.maximum((z - muz) * lax.rsqrt(varz + EPS), 0.0)


def _run_branch(geo, cent, feat, w1t, w2pt, b2p, S, mbr):
    bm = 4096
    grid = (mbr // bm,)
    q = bm // S
    acc1 = pl.pallas_call(
        functools.partial(_pass1_body, S=S, bm=bm),
        grid=grid,
        in_specs=[
            pl.BlockSpec((bm, 4), lambda i: (i, 0)),
            pl.BlockSpec((q, 3), lambda i: (i, 0)),
            pl.BlockSpec((10, CMID), lambda i: (0, 0)),
        ],
        out_specs=pl.BlockSpec((8, 128), lambda i: (0, 0)),
        out_shape=jax.ShapeDtypeStruct((8, 128), jnp.float32),
    )(geo, cent, w1t)
    m, accy = pl.pallas_call(
        functools.partial(_pass2_body, S=S, bm=bm, mbr=mbr),
        grid=grid,
        in_specs=[
            pl.BlockSpec((bm, 4), lambda i: (i, 0)),
            pl.BlockSpec((q, 3), lambda i: (i, 0)),
            pl.BlockSpec((bm, CF), lambda i: (i, 0)),
            pl.BlockSpec((10, CMID), lambda i: (0, 0)),
            pl.BlockSpec((8, 128), lambda i: (0, 0)),
            pl.BlockSpec((CMID, CIN), lambda i: (0, 0)),
            pl.BlockSpec((1, CIN), lambda i: (0, 0)),
        ],
        out_specs=[
            pl.BlockSpec((q, 128), lambda i: (i, 0)),
            pl.BlockSpec((8, 128), lambda i: (0, 0)),
        ],
        out_shape=[
            jax.ShapeDtypeStruct((B * P, 128), jnp.float32),
            jax.ShapeDtypeStruct((8, 128), jnp.float32),
        ],
    )(geo, cent, feat, w1t, acc1, w2pt, b2p)
    return m, accy


def kernel(xyz, features, w1, b1, w2, b2, wcr, bcr):
    xt = jnp.transpose(xyz, (0, 2, 1))
    ox3, oy3, oz3 = _run_fps(xt)
    new_xyz = jnp.concatenate([ox3, oy3, oz3], axis=-1)

    planes = (xt[:, 0], xt[:, 1], xt[:, 2])
    cplanes = (ox3[..., 0], oy3[..., 0], oz3[..., 0])
    idx1, geo1, idx2, geo2 = _run_ball_query(planes, cplanes)

    table = jnp.transpose(features, (0, 2, 1)).reshape(B * N, CF)
    feat1, feat2 = _run_gather(table, idx1.reshape(M1), idx2.reshape(M2))

    perm = jnp.concatenate(
        [jnp.arange(3, CIN, dtype=jnp.int32),
         jnp.arange(0, 3, dtype=jnp.int32)])
    w1t = jnp.transpose(w1)
    w2pt = jnp.transpose(w2[perm, :])
    b2p = b2[perm].reshape(1, CIN)
    wcrt = jnp.transpose(wcr[:, perm])

    cent = new_xyz.reshape(B * P, 3)
    m1, ay1 = _run_branch(geo1.reshape(M1, 4), cent, feat1,
                          w1t, w2pt, b2p, S1, M1)
    m2, ay2 = _run_branch(geo2.reshape(M2, 4), cent, feat2,
                          w1t, w2pt, b2p, S2, M2)

    o1, o2 = pl.pallas_call(
        _pass3_body,
        out_shape=[jax.ShapeDtypeStruct((B * P, COUT), jnp.float32)] * 2,
    )(m1, ay1, m2, ay2, wcrt)

    f1 = o1.reshape(B, P, COUT).transpose(0, 2, 1)
    f2 = o2.reshape(B, P, COUT).transpose(0, 2, 1)
    return (new_xyz, jnp.concatenate([f1, f2], axis=1))


# speedup vs baseline: 2.0714x; 1.1106x over previous
import functools

import jax
import jax.numpy as jnp
from jax import lax
from jax.experimental import pallas as pl
from jax.experimental.pallas import tpu as pltpu
from jax.experimental.pallas import tpu_sc as plsc

B = 4
N = 8192
P = 1024
R1, R2 = 0.1, 0.2
S1, S2 = 16, 32
CF = 64
CIN = CF + 3
CMID = 32
COUT = 128
EPS = 1e-5
M1 = B * P * S1
M2 = B * P * S2
NSUB = 32
CPW = P // (NSUB // B)
NSTEP = N // 16


def _fps_body(x_ref, y_ref, z_ref, ox_ref, oy_ref, oz_ref, dists_ref):
    gidx = (lax.broadcasted_iota(jnp.int32, (B, 8, N // 8), 1) * (N // 8)
            + lax.broadcasted_iota(jnp.int32, (B, 8, N // 8), 2))

    dists_ref[...] = jnp.full((B, 8, N // 8), 1e10, jnp.float32)
    zf = jnp.zeros((B, 8, N // 8), jnp.float32)

    def extract(iv):
        m = gidx == iv
        lx = jnp.sum(jnp.sum(jnp.where(m, x_ref[...], zf), axis=2,
                             keepdims=True), axis=1, keepdims=True)
        ly = jnp.sum(jnp.sum(jnp.where(m, y_ref[...], zf), axis=2,
                             keepdims=True), axis=1, keepdims=True)
        lz = jnp.sum(jnp.sum(jnp.where(m, z_ref[...], zf), axis=2,
                             keepdims=True), axis=1, keepdims=True)
        return lx, ly, lz

    def store(i, lx, ly, lz):
        ox_ref[:, pl.ds(i, 1), :] = lx
        oy_ref[:, pl.ds(i, 1), :] = ly
        oz_ref[:, pl.ds(i, 1), :] = lz

    lx, ly, lz = extract(jnp.zeros((B, 1, 1), jnp.int32))
    store(0, lx, ly, lz)

    def body(i, carry):
        lx, ly, lz = carry
        dx = x_ref[...] - lx
        dy = y_ref[...] - ly
        dz = z_ref[...] - lz
        d2 = dx * dx + dy * dy + dz * dz
        dm = jnp.minimum(dists_ref[...], d2)
        dists_ref[...] = dm
        mx = jnp.max(jnp.max(dm, axis=2, keepdims=True), axis=1,
                     keepdims=True)
        cand = jnp.where(dm == mx, gidx, N)
        iv = jnp.min(jnp.min(cand, axis=2, keepdims=True), axis=1,
                     keepdims=True)
        lx, ly, lz = extract(iv)
        store(i, lx, ly, lz)
        return (lx, ly, lz)

    lax.fori_loop(1, P, body, (lx, ly, lz))


def _run_fps(xt):
    x = xt[:, 0, :].reshape(B, 8, N // 8)
    y = xt[:, 1, :].reshape(B, 8, N // 8)
    z = xt[:, 2, :].reshape(B, 8, N // 8)
    out = pl.pallas_call(
        _fps_body,
        out_shape=[jax.ShapeDtypeStruct((B, P, 1), jnp.float32)] * 3,
        scratch_shapes=[pltpu.VMEM((B, 8, N // 8), jnp.float32)],
    )(x, y, z)
    return out


def _bq_body(xp_hbm, yp_hbm, zp_hbm, cxp_hbm, cyp_hbm, czp_hbm,
             i1_hbm, g1_hbm, i2_hbm, g2_hbm,
             xv, yv, zv, cxv, cyv, czv, o1i, o2i, o1g, o2g,
             t1i, t1d, t2i, t2d, cntv, d2b):
    w = lax.axis_index("c") * 16 + lax.axis_index("s")
    b = w // (NSUB // B)
    p0 = (w % (NSUB // B)) * CPW
    base = b * N

    pltpu.sync_copy(xp_hbm.at[b], xv)
    pltpu.sync_copy(yp_hbm.at[b], yv)
    pltpu.sync_copy(zp_hbm.at[b], zv)
    pltpu.sync_copy(cxp_hbm.at[b, pl.ds(p0, CPW)], cxv.at[pl.ds(0, CPW)])
    pltpu.sync_copy(cyp_hbm.at[b, pl.ds(p0, CPW)], cyv.at[pl.ds(0, CPW)])
    pltpu.sync_copy(czp_hbm.at[b, pl.ds(p0, CPW)], czv.at[pl.ds(0, CPW)])

    lane = lax.broadcasted_iota(jnp.int32, (16,), 0)
    r1sq = jnp.float32(R1 * R1)
    r2sq = jnp.float32(R2 * R2)

    def centroid(c, _):
        cx = cxv[pl.ds(c, 16)][0]
        cy = cyv[pl.ds(c, 16)][0]
        cz = czv[pl.ds(c, 16)][0]
        x0 = xv[pl.ds(0, 16)][0]
        y0 = yv[pl.ds(0, 16)][0]
        z0 = zv[pl.ds(0, 16)][0]
        d20 = ((x0 - cx) * (x0 - cx) + (y0 - cy) * (y0 - cy)
               + (z0 - cz) * (z0 - cz))
        t1i[pl.ds(0, 16)] = jnp.full((16,), base, jnp.int32)
        t1d[pl.ds(0, 16)] = jnp.full((16,), d20, jnp.float32)
        t2i[pl.ds(0, 16)] = jnp.full((16,), base, jnp.int32)
        t2d[pl.ds(0, 16)] = jnp.full((16,), d20, jnp.float32)

        zero16 = jnp.zeros((16,), jnp.int32)
        cntv[pl.ds(0, 16)] = zero16
        cntv[pl.ds(16, 16)] = zero16
        cntv[pl.ds(32, 16)] = zero16
        cntv[pl.ds(48, 16)] = zero16

        def branch(d2s, off4, tbi, tbd, coff, doff, rsq, cap):
            cv = cntv[pl.ds(coff, 16)]
            for k in range(16):
                d2 = d2s[k]
                gi = (base + off4 + 16 * k) + lane
                m = (d2 < rsq) & (cv < cap)
                v = m.astype(jnp.int32)
                inc = plsc.cumsum(v)
                plsc.store_scatter(tbi, [cv + (inc - v)], gi, mask=m)
                plsc.store_scatter(tbd, [cv + (inc - v)], d2, mask=m)
                cv = cv + plsc.all_reduce_population_count(m)
            cntv[pl.ds(coff, 16)] = cv
            cntv[pl.ds(doff, 16)] = (cv >= cap).astype(jnp.int32)

        def chunk(jj, _):
            off4 = jj * 256
            d1 = cntv[pl.ds(32, 16)][0]
            d2f = cntv[pl.ds(48, 16)][0]

            @pl.when(d1 + d2f < 2)
            def _():
                d2s = []
                for k in range(16):
                    xs = xv[pl.ds(off4 + 16 * k, 16)]
                    ys = yv[pl.ds(off4 + 16 * k, 16)]
                    zs = zv[pl.ds(off4 + 16 * k, 16)]
                    dx = xs - cx
                    dy = ys - cy
                    dz = zs - cz
                    d2s.append(dx * dx + dy * dy + dz * dz)

                @pl.when(d1 == 0)
                def _():
                    branch(d2s, off4, t1i, t1d, 0, 32, r1sq, S1)

                @pl.when(d2f == 0)
                def _():
                    branch(d2s, off4, t2i, t2d, 16, 48, r2sq, S2)

            return 0

        lax.fori_loop(0, NSTEP // 16, chunk, 0)
        c1 = jnp.minimum(cntv[pl.ds(0, 16)][0], S1)
        c2 = jnp.minimum(cntv[pl.ds(16, 16)][0], S2)

        cc = jnp.full((16,), c, jnp.int32)
        f_i = t1i[pl.ds(0, 16)][0]
        f_d = t1d[pl.ds(0, 16)][0]
        vi = jnp.where(lane >= c1, f_i, t1i[pl.ds(0, 16)])
        vd = jnp.where(lane >= c1, f_d, t1d[pl.ds(0, 16)])
        o1i[c, :] = vi
        loc = vi - base
        gx = plsc.load_gather(xv, [loc])
        gy = plsc.load_gather(yv, [loc])
        gz = plsc.load_gather(zv, [loc])
        plsc.store_scatter(o1g, [cc, lane, jnp.full((16,), 0, jnp.int32)], gx)
        plsc.store_scatter(o1g, [cc, lane, jnp.full((16,), 1, jnp.int32)], gy)
        plsc.store_scatter(o1g, [cc, lane, jnp.full((16,), 2, jnp.int32)], gz)
        plsc.store_scatter(o1g, [cc, lane, jnp.full((16,), 3, jnp.int32)], vd)
        f_i2 = t2i[pl.ds(0, 16)][0]
        f_d2 = t2d[pl.ds(0, 16)][0]
        for k in range(2):
            lk = lane + (16 * k)
            vi = jnp.where(lk >= c2, f_i2, t2i[pl.ds(16 * k, 16)])
            vd = jnp.where(lk >= c2, f_d2, t2d[pl.ds(16 * k, 16)])
            o2i[c, pl.ds(16 * k, 16)] = vi
            loc = vi - base
            gx = plsc.load_gather(xv, [loc])
            gy = plsc.load_gather(yv, [loc])
            gz = plsc.load_gather(zv, [loc])
            plsc.store_scatter(o2g, [cc, lk, jnp.full((16,), 0, jnp.int32)], gx)
            plsc.store_scatter(o2g, [cc, lk, jnp.full((16,), 1, jnp.int32)], gy)
            plsc.store_scatter(o2g, [cc, lk, jnp.full((16,), 2, jnp.int32)], gz)
            plsc.store_scatter(o2g, [cc, lk, jnp.full((16,), 3, jnp.int32)], vd)
        return 0

    lax.fori_loop(0, CPW, centroid, 0)

    pltpu.sync_copy(o1i, i1_hbm.at[b, pl.ds(p0, CPW)])
    pltpu.sync_copy(o2i, i2_hbm.at[b, pl.ds(p0, CPW)])
    pltpu.sync_copy(o1g, g1_hbm.at[b, pl.ds(p0, CPW)])
    pltpu.sync_copy(o2g, g2_hbm.at[b, pl.ds(p0, CPW)])


def _run_ball_query(planes, cplanes):
    mesh = plsc.VectorSubcoreMesh(core_axis_name="c", subcore_axis_name="s",
                                   num_cores=2, num_subcores=16)
    f = pl.kernel(
        _bq_body,
        out_type=[
            pltpu.HBM((B, P, S1), jnp.int32),
            pltpu.HBM((B, P, S1, 4), jnp.float32),
            pltpu.HBM((B, P, S2), jnp.int32),
            pltpu.HBM((B, P, S2, 4), jnp.float32),
        ],
        mesh=mesh,
        compiler_params=pltpu.CompilerParams(needs_layout_passes=False,
                                             use_tc_tiling_on_sc=False),
        scratch_types=[
            pltpu.VMEM((N,), jnp.float32),
            pltpu.VMEM((N,), jnp.float32),
            pltpu.VMEM((N,), jnp.float32),
            pltpu.VMEM((CPW + 16,), jnp.float32),
            pltpu.VMEM((CPW + 16,), jnp.float32),
            pltpu.VMEM((CPW + 16,), jnp.float32),
            pltpu.VMEM((CPW, S1), jnp.int32),
            pltpu.VMEM((CPW, S2), jnp.int32),
            pltpu.VMEM((CPW, S1, 4), jnp.float32),
            pltpu.VMEM((CPW, S2, 4), jnp.float32),
            pltpu.VMEM((48,), jnp.int32),
            pltpu.VMEM((48,), jnp.float32),
            pltpu.VMEM((48,), jnp.int32),
            pltpu.VMEM((48,), jnp.float32),
            pltpu.VMEM((64,), jnp.int32),
            pltpu.VMEM((64,), jnp.float32),
        ],
    )
    return f(*planes, *cplanes)


def _gather_body(table_hbm, i1_hbm, i2_hbm, g1_hbm, g2_hbm,
                 idxv, buf, sem):
    w = lax.axis_index("c") * 16 + lax.axis_index("s")

    def run(idx_hbm, out_hbm, rows_per_sub):
        nchunk = rows_per_sub // 128
        r0 = w * rows_per_sub

        def chunk(c, _):
            rb = r0 + c * 128
            pltpu.sync_copy(idx_hbm.at[pl.ds(rb, 128)], idxv)
            pltpu.async_copy(table_hbm.at[idxv], buf, sem).wait()
            pltpu.sync_copy(buf, out_hbm.at[pl.ds(rb, 128)])
            return 0

        lax.fori_loop(0, nchunk, chunk, 0)

    run(i1_hbm, g1_hbm, M1 // NSUB)
    run(i2_hbm, g2_hbm, M2 // NSUB)


def _run_gather(table, idx1f, idx2f):
    mesh = plsc.VectorSubcoreMesh(core_axis_name="c", subcore_axis_name="s",
                                   num_cores=2, num_subcores=16)
    f = pl.kernel(
        _gather_body,
        out_type=[
            pltpu.HBM((M1, CF), jnp.float32),
            pltpu.HBM((M2, CF), jnp.float32),
        ],
        mesh=mesh,
        compiler_params=pltpu.CompilerParams(needs_layout_passes=False,
                                             use_tc_tiling_on_sc=False),
        scratch_types=[
            pltpu.VMEM((128,), jnp.int32),
            pltpu.VMEM((128, CF), jnp.float32),
            pltpu.SemaphoreType.DMA,
        ],
    )
    return f(table, idx1f, idx2f)


def _geom_h1(geo, cent, w1t, S, bm):
    q = bm // S
    geo3 = geo.reshape(q, S, 4)
    raw = geo3[:, :, 0:3]
    d2 = geo3[:, :, 3:4]
    delta = raw - cent.reshape(q, 1, 3)
    xi = jnp.broadcast_to(geo3[:, 0:1, 0:3], (q, S, 3))
    dist = jnp.sqrt(d2 + 1e-12)
    h10 = jnp.concatenate([dist, xi, raw, delta], axis=-1).reshape(bm, 10)
    h1 = jnp.dot(h10, w1t, preferred_element_type=jnp.float32)
    return h1, delta


def _pass1_body(geo_ref, cent_ref, w1t_ref, acc_ref, *, S, bm):
    h1, _ = _geom_h1(geo_ref[...], cent_ref[...], w1t_ref[...], S, bm)

    @pl.when(pl.program_id(0) == 0)
    def _():
        acc_ref[...] = jnp.zeros((8, 128), jnp.float32)

    acc_ref[0:1, 0:CMID] += jnp.sum(h1, axis=0, keepdims=True)
    acc_ref[1:2, 0:CMID] += jnp.sum(h1 * h1, axis=0, keepdims=True)


def _pass2_body(geo_ref, cent_ref, feat_ref, w1t_ref, acc1_ref, w2pt_ref,
                b2p_ref, m_ref, accy_ref, *, S, bm, mbr):
    q = bm // S
    h1, delta = _geom_h1(geo_ref[...], cent_ref[...], w1t_ref[...], S, bm)
    s1 = acc1_ref[0:1, 0:CMID]
    q1 = acc1_ref[1:2, 0:CMID]
    mu1 = s1 * (1.0 / mbr)
    var1 = q1 * (1.0 / mbr) - mu1 * mu1
    rs1 = lax.rsqrt(var1 + EPS)
    h = jnp.maximum((h1 - mu1) * rs1, 0.0)
    h2 = jnp.dot(h, w2pt_ref[...], preferred_element_type=jnp.float32)
    h2 = h2 + b2p_ref[...]
    x = jnp.concatenate([feat_ref[...].reshape(q, S, CF), delta], axis=-1)
    y3 = h2.reshape(q, S, CIN) * x
    y2 = y3.reshape(bm, CIN)
    m = jnp.max(y3, axis=1)
    m_ref[...] = jnp.concatenate(
        [m, jnp.zeros((q, 128 - CIN), jnp.float32)], axis=-1)

    @pl.when(pl.program_id(0) == 0)
    def _():
        accy_ref[...] = jnp.zeros((8, 128), jnp.float32)

    accy_ref[0:1, 0:CIN] += jnp.sum(y2, axis=0, keepdims=True)
    accy_ref[1:2, 0:CIN] += jnp.sum(y2 * y2, axis=0, keepdims=True)


def _pass3_body(m1_ref, a1_ref, m2_ref, a2_ref, wcrt_ref, o1_ref, o2_ref):
    for m_ref, a_ref, o_ref, mbr in ((m1_ref, a1_ref, o1_ref, M1),
                                     (m2_ref, a2_ref, o2_ref, M2)):
        sy = a_ref[0:1, 0:CIN]
        qy = a_ref[1:2, 0:CIN]
        mu = sy * (1.0 / mbr)
        var = qy * (1.0 / mbr) - mu * mu
        rs = lax.rsqrt(var + EPS)
        x3 = jnp.maximum((m_ref[...][:, 0:CIN] - mu) * rs, 0.0)
        z = jnp.dot(x3, wcrt_ref[...], preferred_element_type=jnp.float32)
        muz = jnp.sum(z, axis=0, keepdims=True) * (1.0 / (B * P))
        varz = jnp.sum(z * z, axis=0, keepdims=True) * (1.0 / (B * P)) \
            - muz * muz
        o_ref[...] = jnp.maximum((z - muz) * lax.rsqrt(varz + EPS), 0.0)


def _run_branch(geo, cent, feat, w1t, w2pt, b2p, S, mbr):
    bm = 4096
    grid = (mbr // bm,)
    q = bm // S
    acc1 = pl.pallas_call(
        functools.partial(_pass1_body, S=S, bm=bm),
        grid=grid,
        in_specs=[
            pl.BlockSpec((bm, 4), lambda i: (i, 0)),
            pl.BlockSpec((q, 3), lambda i: (i, 0)),
            pl.BlockSpec((10, CMID), lambda i: (0, 0)),
        ],
        out_specs=pl.BlockSpec((8, 128), lambda i: (0, 0)),
        out_shape=jax.ShapeDtypeStruct((8, 128), jnp.float32),
    )(geo, cent, w1t)
    m, accy = pl.pallas_call(
        functools.partial(_pass2_body, S=S, bm=bm, mbr=mbr),
        grid=grid,
        in_specs=[
            pl.BlockSpec((bm, 4), lambda i: (i, 0)),
            pl.BlockSpec((q, 3), lambda i: (i, 0)),
            pl.BlockSpec((bm, CF), lambda i: (i, 0)),
            pl.BlockSpec((10, CMID), lambda i: (0, 0)),
            pl.BlockSpec((8, 128), lambda i: (0, 0)),
            pl.BlockSpec((CMID, CIN), lambda i: (0, 0)),
            pl.BlockSpec((1, CIN), lambda i: (0, 0)),
        ],
        out_specs=[
            pl.BlockSpec((q, 128), lambda i: (i, 0)),
            pl.BlockSpec((8, 128), lambda i: (0, 0)),
        ],
        out_shape=[
            jax.ShapeDtypeStruct((B * P, 128), jnp.float32),
            jax.ShapeDtypeStruct((8, 128), jnp.float32),
        ],
    )(geo, cent, feat, w1t, acc1, w2pt, b2p)
    return m, accy


def kernel(xyz, features, w1, b1, w2, b2, wcr, bcr):
    xt = jnp.transpose(xyz, (0, 2, 1))
    ox3, oy3, oz3 = _run_fps(xt)
    new_xyz = jnp.concatenate([ox3, oy3, oz3], axis=-1)

    planes = (xt[:, 0], xt[:, 1], xt[:, 2])
    cplanes = (ox3[..., 0], oy3[..., 0], oz3[..., 0])
    idx1, geo1, idx2, geo2 = _run_ball_query(planes, cplanes)

    table = jnp.transpose(features, (0, 2, 1)).reshape(B * N, CF)
    feat1, feat2 = _run_gather(table, idx1.reshape(M1), idx2.reshape(M2))

    perm = jnp.concatenate(
        [jnp.arange(3, CIN, dtype=jnp.int32),
         jnp.arange(0, 3, dtype=jnp.int32)])
    w1t = jnp.transpose(w1)
    w2pt = jnp.transpose(w2[perm, :])
    b2p = b2[perm].reshape(1, CIN)
    wcrt = jnp.transpose(wcr[:, perm])

    cent = new_xyz.reshape(B * P, 3)
    m1, ay1 = _run_branch(geo1.reshape(M1, 4), cent, feat1,
                          w1t, w2pt, b2p, S1, M1)
    m2, ay2 = _run_branch(geo2.reshape(M2, 4), cent, feat2,
                          w1t, w2pt, b2p, S2, M2)

    o1, o2 = pl.pallas_call(
        _pass3_body,
        out_shape=[jax.ShapeDtypeStruct((B * P, COUT), jnp.float32)] * 2,
    )(m1, ay1, m2, ay2, wcrt)

    f1 = o1.reshape(B, P, COUT).transpose(0, 2, 1)
    f2 = o2.reshape(B, P, COUT).transpose(0, 2, 1)
    return (new_xyz, jnp.concatenate([f1, f2], axis=1))


# speedup vs baseline: 2.1603x; 1.0429x over previous
import functools

import jax
import jax.numpy as jnp
from jax import lax
from jax.experimental import pallas as pl
from jax.experimental.pallas import tpu as pltpu
from jax.experimental.pallas import tpu_sc as plsc

B = 4
N = 8192
P = 1024
R1, R2 = 0.1, 0.2
S1, S2 = 16, 32
CF = 64
CIN = CF + 3
CMID = 32
COUT = 128
EPS = 1e-5
M1 = B * P * S1
M2 = B * P * S2
NSUB = 32
CPW = P // (NSUB // B)
NSTEP = N // 16


def _fps_body(x_ref, y_ref, z_ref, ox_ref, oy_ref, oz_ref, dists_ref):
    gidx = (lax.broadcasted_iota(jnp.int32, (B, 8, N // 8), 1) * (N // 8)
            + lax.broadcasted_iota(jnp.int32, (B, 8, N // 8), 2))

    dists_ref[...] = jnp.full((B, 8, N // 8), 1e10, jnp.float32)
    zf = jnp.zeros((B, 8, N // 8), jnp.float32)

    def extract(iv):
        m = gidx == iv
        lx = jnp.sum(jnp.sum(jnp.where(m, x_ref[...], zf), axis=2,
                             keepdims=True), axis=1, keepdims=True)
        ly = jnp.sum(jnp.sum(jnp.where(m, y_ref[...], zf), axis=2,
                             keepdims=True), axis=1, keepdims=True)
        lz = jnp.sum(jnp.sum(jnp.where(m, z_ref[...], zf), axis=2,
                             keepdims=True), axis=1, keepdims=True)
        return lx, ly, lz

    def store(i, lx, ly, lz):
        ox_ref[:, pl.ds(i, 1), :] = lx
        oy_ref[:, pl.ds(i, 1), :] = ly
        oz_ref[:, pl.ds(i, 1), :] = lz

    lx, ly, lz = extract(jnp.zeros((B, 1, 1), jnp.int32))
    store(0, lx, ly, lz)

    def body(i, carry):
        lx, ly, lz = carry
        dx = x_ref[...] - lx
        dy = y_ref[...] - ly
        dz = z_ref[...] - lz
        d2 = dx * dx + dy * dy + dz * dz
        dm = jnp.minimum(dists_ref[...], d2)
        dists_ref[...] = dm
        mx = jnp.max(jnp.max(dm, axis=2, keepdims=True), axis=1,
                     keepdims=True)
        cand = jnp.where(dm == mx, gidx, N)
        iv = jnp.min(jnp.min(cand, axis=2, keepdims=True), axis=1,
                     keepdims=True)
        lx, ly, lz = extract(iv)
        store(i, lx, ly, lz)
        return (lx, ly, lz)

    lax.fori_loop(1, P, body, (lx, ly, lz))


def _run_fps(xt):
    x = xt[:, 0, :].reshape(B, 8, N // 8)
    y = xt[:, 1, :].reshape(B, 8, N // 8)
    z = xt[:, 2, :].reshape(B, 8, N // 8)
    out = pl.pallas_call(
        _fps_body,
        out_shape=[jax.ShapeDtypeStruct((B, P, 1), jnp.float32)] * 3,
        scratch_shapes=[pltpu.VMEM((B, 8, N // 8), jnp.float32)],
    )(x, y, z)
    return out


def _bq_body(xp_hbm, yp_hbm, zp_hbm, cxp_hbm, cyp_hbm, czp_hbm,
             i1_hbm, g1_hbm, i2_hbm, g2_hbm,
             xv, yv, zv, cxv, cyv, czv, o1i, o2i, o1g, o2g,
             t1i, t1d, t2i, t2d, cntv, d2b):
    w = lax.axis_index("c") * 16 + lax.axis_index("s")
    b = w // (NSUB // B)
    p0 = (w % (NSUB // B)) * CPW
    base = b * N

    pltpu.sync_copy(xp_hbm.at[b], xv)
    pltpu.sync_copy(yp_hbm.at[b], yv)
    pltpu.sync_copy(zp_hbm.at[b], zv)
    pltpu.sync_copy(cxp_hbm.at[b, pl.ds(p0, CPW)], cxv.at[pl.ds(0, CPW)])
    pltpu.sync_copy(cyp_hbm.at[b, pl.ds(p0, CPW)], cyv.at[pl.ds(0, CPW)])
    pltpu.sync_copy(czp_hbm.at[b, pl.ds(p0, CPW)], czv.at[pl.ds(0, CPW)])

    lane = lax.broadcasted_iota(jnp.int32, (16,), 0)
    r1sq = jnp.float32(R1 * R1)
    r2sq = jnp.float32(R2 * R2)

    def centroid(c, _):
        cx = cxv[pl.ds(c, 16)][0]
        cy = cyv[pl.ds(c, 16)][0]
        cz = czv[pl.ds(c, 16)][0]
        x0 = xv[pl.ds(0, 16)][0]
        y0 = yv[pl.ds(0, 16)][0]
        z0 = zv[pl.ds(0, 16)][0]
        d20 = ((x0 - cx) * (x0 - cx) + (y0 - cy) * (y0 - cy)
               + (z0 - cz) * (z0 - cz))
        t1i[pl.ds(0, 16)] = jnp.full((16,), base, jnp.int32)
        t1d[pl.ds(0, 16)] = jnp.full((16,), d20, jnp.float32)
        t2i[pl.ds(0, 16)] = jnp.full((16,), base, jnp.int32)
        t2d[pl.ds(0, 16)] = jnp.full((16,), d20, jnp.float32)

        zero16 = jnp.zeros((16,), jnp.int32)
        cntv[pl.ds(0, 16)] = zero16
        cntv[pl.ds(16, 16)] = zero16
        cntv[pl.ds(32, 16)] = zero16
        cntv[pl.ds(48, 16)] = zero16

        def branch(d2s, off4, tbi, tbd, coff, doff, rsq, cap):
            cv = cntv[pl.ds(coff, 16)]
            for k in range(32):
                d2 = d2s[k]
                gi = (base + off4 + 16 * k) + lane
                m = (d2 < rsq) & (cv < cap)
                v = m.astype(jnp.int32)
                inc = plsc.cumsum(v)
                plsc.store_scatter(tbi, [cv + (inc - v)], gi, mask=m)
                plsc.store_scatter(tbd, [cv + (inc - v)], d2, mask=m)
                cv = cv + plsc.all_reduce_population_count(m)
            cntv[pl.ds(coff, 16)] = cv
            cntv[pl.ds(doff, 16)] = (cv >= cap).astype(jnp.int32)

        def chunk(jj, _):
            off4 = jj * 512
            d1 = cntv[pl.ds(32, 16)][0]
            d2f = cntv[pl.ds(48, 16)][0]

            @pl.when(d1 + d2f < 2)
            def _():
                d2s = []
                for k in range(32):
                    xs = xv[pl.ds(off4 + 16 * k, 16)]
                    ys = yv[pl.ds(off4 + 16 * k, 16)]
                    zs = zv[pl.ds(off4 + 16 * k, 16)]
                    dx = xs - cx
                    dy = ys - cy
                    dz = zs - cz
                    d2s.append(dx * dx + dy * dy + dz * dz)

                @pl.when(d1 == 0)
                def _():
                    branch(d2s, off4, t1i, t1d, 0, 32, r1sq, S1)

                @pl.when(d2f == 0)
                def _():
                    branch(d2s, off4, t2i, t2d, 16, 48, r2sq, S2)

            return 0

        lax.fori_loop(0, NSTEP // 32, chunk, 0)
        c1 = jnp.minimum(cntv[pl.ds(0, 16)][0], S1)
        c2 = jnp.minimum(cntv[pl.ds(16, 16)][0], S2)

        cc = jnp.full((16,), c, jnp.int32)
        f_i = t1i[pl.ds(0, 16)][0]
        f_d = t1d[pl.ds(0, 16)][0]
        vi = jnp.where(lane >= c1, f_i, t1i[pl.ds(0, 16)])
        vd = jnp.where(lane >= c1, f_d, t1d[pl.ds(0, 16)])
        o1i[c, :] = vi
        loc = vi - base
        gx = plsc.load_gather(xv, [loc])
        gy = plsc.load_gather(yv, [loc])
        gz = plsc.load_gather(zv, [loc])
        plsc.store_scatter(o1g, [cc, lane, jnp.full((16,), 0, jnp.int32)], gx)
        plsc.store_scatter(o1g, [cc, lane, jnp.full((16,), 1, jnp.int32)], gy)
        plsc.store_scatter(o1g, [cc, lane, jnp.full((16,), 2, jnp.int32)], gz)
        plsc.store_scatter(o1g, [cc, lane, jnp.full((16,), 3, jnp.int32)], vd)
        f_i2 = t2i[pl.ds(0, 16)][0]
        f_d2 = t2d[pl.ds(0, 16)][0]
        for k in range(2):
            lk = lane + (16 * k)
            vi = jnp.where(lk >= c2, f_i2, t2i[pl.ds(16 * k, 16)])
            vd = jnp.where(lk >= c2, f_d2, t2d[pl.ds(16 * k, 16)])
            o2i[c, pl.ds(16 * k, 16)] = vi
            loc = vi - base
            gx = plsc.load_gather(xv, [loc])
            gy = plsc.load_gather(yv, [loc])
            gz = plsc.load_gather(zv, [loc])
            plsc.store_scatter(o2g, [cc, lk, jnp.full((16,), 0, jnp.int32)], gx)
            plsc.store_scatter(o2g, [cc, lk, jnp.full((16,), 1, jnp.int32)], gy)
            plsc.store_scatter(o2g, [cc, lk, jnp.full((16,), 2, jnp.int32)], gz)
            plsc.store_scatter(o2g, [cc, lk, jnp.full((16,), 3, jnp.int32)], vd)
        return 0

    lax.fori_loop(0, CPW, centroid, 0)

    pltpu.sync_copy(o1i, i1_hbm.at[b, pl.ds(p0, CPW)])
    pltpu.sync_copy(o2i, i2_hbm.at[b, pl.ds(p0, CPW)])
    pltpu.sync_copy(o1g, g1_hbm.at[b, pl.ds(p0, CPW)])
    pltpu.sync_copy(o2g, g2_hbm.at[b, pl.ds(p0, CPW)])


def _run_ball_query(planes, cplanes):
    mesh = plsc.VectorSubcoreMesh(core_axis_name="c", subcore_axis_name="s",
                                   num_cores=2, num_subcores=16)
    f = pl.kernel(
        _bq_body,
        out_type=[
            pltpu.HBM((B, P, S1), jnp.int32),
            pltpu.HBM((B, P, S1, 4), jnp.float32),
            pltpu.HBM((B, P, S2), jnp.int32),
            pltpu.HBM((B, P, S2, 4), jnp.float32),
        ],
        mesh=mesh,
        compiler_params=pltpu.CompilerParams(needs_layout_passes=False,
                                             use_tc_tiling_on_sc=False),
        scratch_types=[
            pltpu.VMEM((N,), jnp.float32),
            pltpu.VMEM((N,), jnp.float32),
            pltpu.VMEM((N,), jnp.float32),
            pltpu.VMEM((CPW + 16,), jnp.float32),
            pltpu.VMEM((CPW + 16,), jnp.float32),
            pltpu.VMEM((CPW + 16,), jnp.float32),
            pltpu.VMEM((CPW, S1), jnp.int32),
            pltpu.VMEM((CPW, S2), jnp.int32),
            pltpu.VMEM((CPW, S1, 4), jnp.float32),
            pltpu.VMEM((CPW, S2, 4), jnp.float32),
            pltpu.VMEM((48,), jnp.int32),
            pltpu.VMEM((48,), jnp.float32),
            pltpu.VMEM((48,), jnp.int32),
            pltpu.VMEM((48,), jnp.float32),
            pltpu.VMEM((64,), jnp.int32),
            pltpu.VMEM((64,), jnp.float32),
        ],
    )
    return f(*planes, *cplanes)


def _gather_body(table_hbm, i1_hbm, i2_hbm, g1_hbm, g2_hbm,
                 idxv, buf, sem):
    w = lax.axis_index("c") * 16 + lax.axis_index("s")

    def run(idx_hbm, out_hbm, rows_per_sub):
        nchunk = rows_per_sub // 128
        r0 = w * rows_per_sub

        def chunk(c, _):
            rb = r0 + c * 128
            pltpu.sync_copy(idx_hbm.at[pl.ds(rb, 128)], idxv)
            pltpu.async_copy(table_hbm.at[idxv], buf, sem).wait()
            pltpu.sync_copy(buf, out_hbm.at[pl.ds(rb, 128)])
            return 0

        lax.fori_loop(0, nchunk, chunk, 0)

    run(i1_hbm, g1_hbm, M1 // NSUB)
    run(i2_hbm, g2_hbm, M2 // NSUB)


def _run_gather(table, idx1f, idx2f):
    mesh = plsc.VectorSubcoreMesh(core_axis_name="c", subcore_axis_name="s",
                                   num_cores=2, num_subcores=16)
    f = pl.kernel(
        _gather_body,
        out_type=[
            pltpu.HBM((M1, CF), jnp.float32),
            pltpu.HBM((M2, CF), jnp.float32),
        ],
        mesh=mesh,
        compiler_params=pltpu.CompilerParams(needs_layout_passes=False,
                                             use_tc_tiling_on_sc=False),
        scratch_types=[
            pltpu.VMEM((128,), jnp.int32),
            pltpu.VMEM((128, CF), jnp.float32),
            pltpu.SemaphoreType.DMA,
        ],
    )
    return f(table, idx1f, idx2f)


def _geom_h1(geo, cent, w1t, S, bm):
    q = bm // S
    geo3 = geo.reshape(q, S, 4)
    raw = geo3[:, :, 0:3]
    d2 = geo3[:, :, 3:4]
    delta = raw - cent.reshape(q, 1, 3)
    xi = jnp.broadcast_to(geo3[:, 0:1, 0:3], (q, S, 3))
    dist = jnp.sqrt(d2 + 1e-12)
    h10 = jnp.concatenate([dist, xi, raw, delta], axis=-1).reshape(bm, 10)
    h1 = jnp.dot(h10, w1t, preferred_element_type=jnp.float32)
    return h1, delta


def _pass1_body(geo_ref, cent_ref, w1t_ref, acc_ref, *, S, bm):
    h1, _ = _geom_h1(geo_ref[...], cent_ref[...], w1t_ref[...], S, bm)

    @pl.when(pl.program_id(0) == 0)
    def _():
        acc_ref[...] = jnp.zeros((8, 128), jnp.float32)

    acc_ref[0:1, 0:CMID] += jnp.sum(h1, axis=0, keepdims=True)
    acc_ref[1:2, 0:CMID] += jnp.sum(h1 * h1, axis=0, keepdims=True)


def _pass2_body(geo_ref, cent_ref, feat_ref, w1t_ref, acc1_ref, w2pt_ref,
                b2p_ref, m_ref, accy_ref, *, S, bm, mbr):
    q = bm // S
    h1, delta = _geom_h1(geo_ref[...], cent_ref[...], w1t_ref[...], S, bm)
    s1 = acc1_ref[0:1, 0:CMID]
    q1 = acc1_ref[1:2, 0:CMID]
    mu1 = s1 * (1.0 / mbr)
    var1 = q1 * (1.0 / mbr) - mu1 * mu1
    rs1 = lax.rsqrt(var1 + EPS)
    h = jnp.maximum((h1 - mu1) * rs1, 0.0)
    h2 = jnp.dot(h, w2pt_ref[...], preferred_element_type=jnp.float32)
    h2 = h2 + b2p_ref[...]
    x = jnp.concatenate([feat_ref[...].reshape(q, S, CF), delta], axis=-1)
    y3 = h2.reshape(q, S, CIN) * x
    y2 = y3.reshape(bm, CIN)
    m = jnp.max(y3, axis=1)
    m_ref[...] = jnp.concatenate(
        [m, jnp.zeros((q, 128 - CIN), jnp.float32)], axis=-1)

    @pl.when(pl.program_id(0) == 0)
    def _():
        accy_ref[...] = jnp.zeros((8, 128), jnp.float32)

    accy_ref[0:1, 0:CIN] += jnp.sum(y2, axis=0, keepdims=True)
    accy_ref[1:2, 0:CIN] += jnp.sum(y2 * y2, axis=0, keepdims=True)


def _pass3_body(m1_ref, a1_ref, m2_ref, a2_ref, wcrt_ref, o1_ref, o2_ref):
    for m_ref, a_ref, o_ref, mbr in ((m1_ref, a1_ref, o1_ref, M1),
                                     (m2_ref, a2_ref, o2_ref, M2)):
        sy = a_ref[0:1, 0:CIN]
        qy = a_ref[1:2, 0:CIN]
        mu = sy * (1.0 / mbr)
        var = qy * (1.0 / mbr) - mu * mu
        rs = lax.rsqrt(var + EPS)
        x3 = jnp.maximum((m_ref[...][:, 0:CIN] - mu) * rs, 0.0)
        z = jnp.dot(x3, wcrt_ref[...], preferred_element_type=jnp.float32)
        muz = jnp.sum(z, axis=0, keepdims=True) * (1.0 / (B * P))
        varz = jnp.sum(z * z, axis=0, keepdims=True) * (1.0 / (B * P)) \
            - muz * muz
        o_ref[...] = jnp.maximum((z - muz) * lax.rsqrt(varz + EPS), 0.0)


def _run_branch(geo, cent, feat, w1t, w2pt, b2p, S, mbr):
    bm = 4096
    grid = (mbr // bm,)
    q = bm // S
    acc1 = pl.pallas_call(
        functools.partial(_pass1_body, S=S, bm=bm),
        grid=grid,
        in_specs=[
            pl.BlockSpec((bm, 4), lambda i: (i, 0)),
            pl.BlockSpec((q, 3), lambda i: (i, 0)),
            pl.BlockSpec((10, CMID), lambda i: (0, 0)),
        ],
        out_specs=pl.BlockSpec((8, 128), lambda i: (0, 0)),
        out_shape=jax.ShapeDtypeStruct((8, 128), jnp.float32),
    )(geo, cent, w1t)
    m, accy = pl.pallas_call(
        functools.partial(_pass2_body, S=S, bm=bm, mbr=mbr),
        grid=grid,
        in_specs=[
            pl.BlockSpec((bm, 4), lambda i: (i, 0)),
            pl.BlockSpec((q, 3), lambda i: (i, 0)),
            pl.BlockSpec((bm, CF), lambda i: (i, 0)),
            pl.BlockSpec((10, CMID), lambda i: (0, 0)),
            pl.BlockSpec((8, 128), lambda i: (0, 0)),
            pl.BlockSpec((CMID, CIN), lambda i: (0, 0)),
            pl.BlockSpec((1, CIN), lambda i: (0, 0)),
        ],
        out_specs=[
            pl.BlockSpec((q, 128), lambda i: (i, 0)),
            pl.BlockSpec((8, 128), lambda i: (0, 0)),
        ],
        out_shape=[
            jax.ShapeDtypeStruct((B * P, 128), jnp.float32),
            jax.ShapeDtypeStruct((8, 128), jnp.float32),
        ],
    )(geo, cent, feat, w1t, acc1, w2pt, b2p)
    return m, accy


def kernel(xyz, features, w1, b1, w2, b2, wcr, bcr):
    xt = jnp.transpose(xyz, (0, 2, 1))
    ox3, oy3, oz3 = _run_fps(xt)
    new_xyz = jnp.concatenate([ox3, oy3, oz3], axis=-1)

    planes = (xt[:, 0], xt[:, 1], xt[:, 2])
    cplanes = (ox3[..., 0], oy3[..., 0], oz3[..., 0])
    idx1, geo1, idx2, geo2 = _run_ball_query(planes, cplanes)

    table = jnp.transpose(features, (0, 2, 1)).reshape(B * N, CF)
    feat1, feat2 = _run_gather(table, idx1.reshape(M1), idx2.reshape(M2))

    perm = jnp.concatenate(
        [jnp.arange(3, CIN, dtype=jnp.int32),
         jnp.arange(0, 3, dtype=jnp.int32)])
    w1t = jnp.transpose(w1)
    w2pt = jnp.transpose(w2[perm, :])
    b2p = b2[perm].reshape(1, CIN)
    wcrt = jnp.transpose(wcr[:, perm])

    cent = new_xyz.reshape(B * P, 3)
    m1, ay1 = _run_branch(geo1.reshape(M1, 4), cent, feat1,
                          w1t, w2pt, b2p, S1, M1)
    m2, ay2 = _run_branch(geo2.reshape(M2, 4), cent, feat2,
                          w1t, w2pt, b2p, S2, M2)

    o1, o2 = pl.pallas_call(
        _pass3_body,
        out_shape=[jax.ShapeDtypeStruct((B * P, COUT), jnp.float32)] * 2,
    )(m1, ay1, m2, ay2, wcrt)

    f1 = o1.reshape(B, P, COUT).transpose(0, 2, 1)
    f2 = o2.reshape(B, P, COUT).transpose(0, 2, 1)
    return (new_xyz, jnp.concatenate([f1, f2], axis=1))
